# Initial kernel scaffold; baseline (speedup 1.0000x reference)
import jax
import jax.numpy as jnp
from jax.experimental import pallas as pl


def kernel(x, edge_index, batch, graph_features, params):
    raise NotImplementedError("write your pallas kernel here")


# speedup vs baseline: 5.7355x; 5.7355x over previous
import functools

import jax
import jax.numpy as jnp
from jax.experimental import pallas as pl
from jax.experimental.pallas import tpu as pltpu
from jax.experimental.pallas import tpu_sc as plsc

N = 10000
E = 320000
D = 128
H = 128
G = 64
GF = 32
NLAYERS = 4
BN_SCALE = (1.0 + 1e-5) ** -0.5

NC = 2
NS = 16
NW = NC * NS
K = 128
CH = 80
E_PAD = NW * CH * K
NPAD = 10240
APT = NPAD // NS
RPT = NPAD // NW


def _sc_deg_body(dst_hbm, out_hbm, dst_v, ones_v, zbuf, acc):
    c = jax.lax.axis_index("c")
    s = jax.lax.axis_index("s")
    wid = s * NC + c
    one16 = jnp.ones((16,), jnp.float32)
    zero16 = jnp.zeros((16,), jnp.float32)

    def _fill_ones(i, _):
        ones_v[pl.ds(i * 16, 16)] = one16
        return 0

    jax.lax.fori_loop(0, K // 16, _fill_ones, 0)

    def _fill_zero(i, _):
        zbuf[pl.ds(i * 16, 16)] = zero16
        return 0

    jax.lax.fori_loop(0, APT // 16, _fill_zero, 0)
    pltpu.sync_copy(zbuf, acc.at[pl.ds(s * APT, APT)])
    plsc.subcore_barrier()
    pltpu.sync_copy(dst_hbm.at[wid], dst_v)

    def _step(j, _):
        pltpu.sync_copy(ones_v, acc.at[dst_v.at[j]], add=True)
        return 0

    jax.lax.fori_loop(0, CH, _step, 0)
    plsc.subcore_barrier()
    pltpu.sync_copy(acc.at[pl.ds(s * APT, APT)],
                    out_hbm.at[c, pl.ds(s * APT, APT)])


HH = H // 2


def _sc_scatter_body(ypl_hbm, ypr_hbm, src_hbm, dst_hbm, out_hbm,
                     src_v, dst_v, rows0, rows1, acc, sem0, sem1):
    c = jax.lax.axis_index("c")
    s = jax.lax.axis_index("s")
    wid = s * NC + c
    zero16 = jnp.zeros((16,), jnp.float32)

    def _zrow(r, _):
        for m in range(HH // 16):
            rows0[r, pl.ds(m * 16, 16)] = zero16
        return 0

    pltpu.sync_copy(src_hbm.at[wid], src_v)
    pltpu.sync_copy(dst_hbm.at[wid], dst_v)
    for half, tab in ((0, ypl_hbm), (1, ypr_hbm)):
        jax.lax.fori_loop(0, K, _zrow, 0)
        for q in range(APT // K):
            pltpu.sync_copy(rows0, acc.at[pl.ds(s * APT + q * K, K)])
        plsc.subcore_barrier()

        def _step(g, _, tab=tab):
            j0 = g * 2
            j1 = j0 + 1
            cp0 = pltpu.async_copy(tab.at[src_v.at[j0]], rows0, sem0)
            cp1 = pltpu.async_copy(tab.at[src_v.at[j1]], rows1, sem1)
            cp0.wait()
            pltpu.sync_copy(rows0, acc.at[dst_v.at[j0]], add=True)
            cp1.wait()
            pltpu.sync_copy(rows1, acc.at[dst_v.at[j1]], add=True)
            return 0

        jax.lax.fori_loop(0, CH // 2, _step, 0)
        plsc.subcore_barrier()
        pltpu.sync_copy(acc.at[pl.ds(s * APT, APT)],
                        out_hbm.at[c, half, pl.ds(s * APT, APT)])


def _sc_maxpool_body(h_hbm, b_hbm, out_hbm, hbuf, bseg, mx):
    c = jax.lax.axis_index("c")
    s = jax.lax.axis_index("s")
    wid = s * NC + c
    neg16 = jnp.full((16,), -jnp.inf, jnp.float32)

    def _init(r, _):
        for m in range(H // 16):
            mx[r, pl.ds(m * 16, 16)] = neg16
        return 0

    jax.lax.fori_loop(0, G + 8, _init, 0)
    pltpu.sync_copy(b_hbm.at[pl.ds(wid * RPT, RPT)], bseg)

    def _chunk(q, _):
        pltpu.sync_copy(h_hbm.at[pl.ds(wid * RPT + q * 64, 64)], hbuf)

        def _grp(gi, _):
            sv = bseg[pl.ds(q * 64 + gi * 16, 16)]
            for t in range(16):
                seg = sv[t]
                r = gi * 16 + t
                for m in range(H // 16):
                    sl = pl.ds(m * 16, 16)
                    mx[seg, sl] = jnp.maximum(mx[seg, sl], hbuf[r, sl])
            return 0

        jax.lax.fori_loop(0, 4, _grp, 0)
        return 0

    jax.lax.fori_loop(0, RPT // 64, _chunk, 0)
    pltpu.sync_copy(mx.at[pl.ds(0, G)], out_hbm.at[wid])


@functools.cache
def _sc_kernels():
    mesh = plsc.VectorSubcoreMesh(core_axis_name="c", subcore_axis_name="s",
                                  num_cores=NC, num_subcores=NS)
    deg = pl.kernel(
        _sc_deg_body,
        out_type=jax.ShapeDtypeStruct((NC, NPAD), jnp.float32),
        mesh=mesh,
        scratch_types=[
            pltpu.VMEM((CH, K), jnp.int32),
            pltpu.VMEM((K,), jnp.float32),
            pltpu.VMEM((APT,), jnp.float32),
            pltpu.VMEM_SHARED((NPAD,), jnp.float32),
        ],
    )
    scatter = pl.kernel(
        _sc_scatter_body,
        out_type=jax.ShapeDtypeStruct((NC, 2, NPAD, HH), jnp.float32),
        mesh=mesh,
        compiler_params=pltpu.CompilerParams(use_tc_tiling_on_sc=False),
        scratch_types=[
            pltpu.VMEM((CH, K), jnp.int32),
            pltpu.VMEM((CH, K), jnp.int32),
            pltpu.VMEM((K, HH), jnp.float32),
            pltpu.VMEM((K, HH), jnp.float32),
            pltpu.VMEM_SHARED((NPAD, HH), jnp.float32),
            pltpu.SemaphoreType.DMA,
            pltpu.SemaphoreType.DMA,
        ],
    )
    maxpool = pl.kernel(
        _sc_maxpool_body,
        out_type=jax.ShapeDtypeStruct((NW, G, H), jnp.float32),
        mesh=mesh,
        scratch_types=[
            pltpu.VMEM((64, H), jnp.float32),
            pltpu.VMEM((RPT,), jnp.int32),
            pltpu.VMEM((G + 8, H), jnp.float32),
        ],
    )
    return deg, scatter, maxpool


def _sc_deg(dst3):
    return _sc_kernels()[0](dst3)


def _sc_scatter(ypl, ypr, src3, dst3):
    return _sc_kernels()[1](ypl, ypr, src3, dst3)


def _sc_maxpool(h, batchp):
    return _sc_kernels()[2](h, batchp)


BLK = 1024
GRID = NPAD // BLK


def _layer0_body(x_ref, d0_ref, d1_ref, w_ref, ypl_ref, ypr_ref, dinv_ref):
    dinv = jax.lax.rsqrt(1.0 + d0_ref[...] + d1_ref[...])
    dinv_ref[...] = dinv
    y = jnp.dot(x_ref[...], w_ref[...], preferred_element_type=jnp.float32)
    yp = y * dinv
    ypl_ref[...] = yp[:, :HH]
    ypr_ref[...] = yp[:, HH:]


_tc_layer0 = pl.pallas_call(
    _layer0_body,
    grid=(GRID,),
    in_specs=[
        pl.BlockSpec((BLK, D), lambda i: (i, 0)),
        pl.BlockSpec((BLK, 1), lambda i: (i, 0)),
        pl.BlockSpec((BLK, 1), lambda i: (i, 0)),
        pl.BlockSpec((D, H), lambda i: (0, 0)),
    ],
    out_specs=[
        pl.BlockSpec((BLK, HH), lambda i: (i, 0)),
        pl.BlockSpec((BLK, HH), lambda i: (i, 0)),
        pl.BlockSpec((BLK, 1), lambda i: (i, 0)),
    ],
    out_shape=[
        jax.ShapeDtypeStruct((NPAD, HH), jnp.float32),
        jax.ShapeDtypeStruct((NPAD, HH), jnp.float32),
        jax.ShapeDtypeStruct((NPAD, 1), jnp.float32),
    ],
)


def _state(p_ref, yppl_ref, yppr_ref, dinv_ref, b_ref, g_ref, bb_ref):
    accl = p_ref[0] + p_ref[2] + yppl_ref[...]
    accr = p_ref[1] + p_ref[3] + yppr_ref[...]
    agg = (dinv_ref[...] * jnp.concatenate([accl, accr], axis=1)
           + b_ref[...])
    return jnp.maximum(agg * BN_SCALE * g_ref[...] + bb_ref[...], 0.0)


def _layer_body(p_ref, yppl_ref, yppr_ref, dinv_ref, b_ref, g_ref, bb_ref,
                w_ref, ypl_ref, ypr_ref):
    st = _state(p_ref, yppl_ref, yppr_ref, dinv_ref, b_ref, g_ref, bb_ref)
    y = jnp.dot(st, w_ref[...], preferred_element_type=jnp.float32)
    yp = y * dinv_ref[...]
    ypl_ref[...] = yp[:, :HH]
    ypr_ref[...] = yp[:, HH:]


_tc_layer = pl.pallas_call(
    _layer_body,
    grid=(GRID,),
    in_specs=[
        pl.BlockSpec((4, BLK, HH), lambda i: (0, i, 0)),
        pl.BlockSpec((BLK, HH), lambda i: (i, 0)),
        pl.BlockSpec((BLK, HH), lambda i: (i, 0)),
        pl.BlockSpec((BLK, 1), lambda i: (i, 0)),
        pl.BlockSpec((1, H), lambda i: (0, 0)),
        pl.BlockSpec((1, H), lambda i: (0, 0)),
        pl.BlockSpec((1, H), lambda i: (0, 0)),
        pl.BlockSpec((H, H), lambda i: (0, 0)),
    ],
    out_specs=[
        pl.BlockSpec((BLK, HH), lambda i: (i, 0)),
        pl.BlockSpec((BLK, HH), lambda i: (i, 0)),
    ],
    out_shape=[
        jax.ShapeDtypeStruct((NPAD, HH), jnp.float32),
        jax.ShapeDtypeStruct((NPAD, HH), jnp.float32),
    ],
)


def _final_body(p_ref, yppl_ref, yppr_ref, dinv_ref, b_ref, g_ref, bb_ref,
                batch_ref, h_ref, ssum_ref, cnt_ref):
    i = pl.program_id(0)
    h = _state(p_ref, yppl_ref, yppr_ref, dinv_ref, b_ref, g_ref, bb_ref)
    h_ref[...] = h
    oh = (batch_ref[...] == jax.lax.broadcasted_iota(jnp.int32, (BLK, G), 1))
    oh = oh.astype(jnp.float32)
    dn = (((0,), (0,)), ((), ()))
    ps = jax.lax.dot_general(oh, h, dn, preferred_element_type=jnp.float32)
    pc = jax.lax.dot_general(oh, jnp.ones((BLK, H), jnp.float32), dn,
                             preferred_element_type=jnp.float32)

    @pl.when(i == 0)
    def _():
        ssum_ref[...] = ps
        cnt_ref[...] = pc

    @pl.when(i != 0)
    def _():
        ssum_ref[...] += ps
        cnt_ref[...] += pc


_tc_final = pl.pallas_call(
    _final_body,
    grid=(GRID,),
    in_specs=[
        pl.BlockSpec((4, BLK, HH), lambda i: (0, i, 0)),
        pl.BlockSpec((BLK, HH), lambda i: (i, 0)),
        pl.BlockSpec((BLK, HH), lambda i: (i, 0)),
        pl.BlockSpec((BLK, 1), lambda i: (i, 0)),
        pl.BlockSpec((1, H), lambda i: (0, 0)),
        pl.BlockSpec((1, H), lambda i: (0, 0)),
        pl.BlockSpec((1, H), lambda i: (0, 0)),
        pl.BlockSpec((BLK, 1), lambda i: (i, 0)),
    ],
    out_specs=[
        pl.BlockSpec((BLK, H), lambda i: (i, 0)),
        pl.BlockSpec((G, H), lambda i: (0, 0)),
        pl.BlockSpec((G, H), lambda i: (0, 0)),
    ],
    out_shape=[
        jax.ShapeDtypeStruct((NPAD, H), jnp.float32),
        jax.ShapeDtypeStruct((G, H), jnp.float32),
        jax.ShapeDtypeStruct((G, H), jnp.float32),
    ],
)


def _head_body(ssum_ref, cnt_ref, maxp_ref, gfin_ref,
               gw1, gb1, gg1, gbb1, gw2, gb2, gg2, gbb2,
               fw1, fb1, fg1, fbb1, fw2, fb2, fg2, fbb2, fw3, fb3,
               z_ref):
    cnt = cnt_ref[:, 0:1]
    ssum = ssum_ref[...]
    smax = jnp.max(maxp_ref[...], axis=0)
    x1 = ssum / jnp.maximum(cnt, 1.0)
    x2 = jnp.where(cnt > 0.0, smax, 0.0)

    def bnrelu(t, g, bb):
        return jnp.maximum(t * BN_SCALE * g[...] + bb[...], 0.0)

    gf = jnp.dot(gfin_ref[...], gw1[...],
                 preferred_element_type=jnp.float32) + gb1[...]
    gf = bnrelu(gf, gg1, gbb1)
    gf = jnp.dot(gf, gw2[...], preferred_element_type=jnp.float32) + gb2[...]
    gf = bnrelu(gf, gg2, gbb2)
    fused = jnp.concatenate([x1, x2, ssum, gf], axis=1)
    z = jnp.dot(fused, fw1[...], preferred_element_type=jnp.float32) + fb1[...]
    z = bnrelu(z, fg1, fbb1)
    z = jnp.dot(z, fw2[...], preferred_element_type=jnp.float32) + fb2[...]
    z = bnrelu(z, fg2, fbb2)
    z_ref[...] = jnp.dot(z, fw3[...],
                         preferred_element_type=jnp.float32) + fb3[...]


_tc_head = pl.pallas_call(
    _head_body,
    out_shape=jax.ShapeDtypeStruct((G, 1), jnp.float32),
)


def kernel(x, edge_index, batch, graph_features, params):
    src = edge_index[0].astype(jnp.int32)
    dst = edge_index[1].astype(jnp.int32)
    src3 = jnp.concatenate(
        [src, jnp.zeros((E_PAD - E,), jnp.int32)]).reshape(NW, CH, K)
    dst3 = jnp.concatenate(
        [dst, jnp.full((E_PAD - E,), N, jnp.int32)]).reshape(NW, CH, K)
    batchp = jnp.concatenate(
        [batch.astype(jnp.int32), jnp.full((NPAD - N,), G, jnp.int32)])
    xp = jnp.pad(x, ((0, NPAD - N), (0, 0)))

    def row(v):
        return v.reshape(1, -1)

    degp = _sc_deg(dst3)
    d0 = degp[0].reshape(NPAD, 1)
    d1 = degp[1].reshape(NPAD, 1)
    ypl, ypr, dinv = _tc_layer0(xp, d0, d1, params["gcn_w0"])
    for l in range(1, NLAYERS):
        p = _sc_scatter(ypl, ypr, src3, dst3).reshape(4, NPAD, HH)
        ypl, ypr = _tc_layer(p, ypl, ypr, dinv,
                             row(params[f"gcn_b{l-1}"]),
                             row(params[f"bn_g{l-1}"]),
                             row(params[f"bn_b{l-1}"]), params[f"gcn_w{l}"])
    p = _sc_scatter(ypl, ypr, src3, dst3).reshape(4, NPAD, HH)
    h, ssum, cnt2 = _tc_final(p, ypl, ypr, dinv,
                              row(params["gcn_b3"]), row(params["bn_g3"]),
                              row(params["bn_b3"]),
                              batchp.reshape(NPAD, 1))
    maxp = _sc_maxpool(h, batchp)
    z = _tc_head(ssum, cnt2, maxp, graph_features,
                 params["gm_w1"], row(params["gm_b1"]),
                 row(params["gm_g1"]), row(params["gm_bb1"]),
                 params["gm_w2"], row(params["gm_b2"]),
                 row(params["gm_g2"]), row(params["gm_bb2"]),
                 params["f_w1"], row(params["f_b1"]),
                 row(params["f_g1"]), row(params["f_bb1"]),
                 params["f_w2"], row(params["f_b2"]),
                 row(params["f_g2"]), row(params["f_bb2"]),
                 params["f_w3"], row(params["f_b3"]))
    return z


# speedup vs baseline: 6.4210x; 1.1195x over previous
import functools

import jax
import jax.numpy as jnp
from jax.experimental import pallas as pl
from jax.experimental.pallas import tpu as pltpu
from jax.experimental.pallas import tpu_sc as plsc

N = 10000
E = 320000
D = 128
H = 128
G = 64
GF = 32
NLAYERS = 4
BN_SCALE = (1.0 + 1e-5) ** -0.5

NC = 2
NS = 16
NW = NC * NS
K = 128
CH = 80
E_PAD = NW * CH * K
NPAD = 10240
APT = NPAD // NS
RPT = NPAD // NW


def _sc_deg_body(dst_hbm, out_hbm, dst_v, ones_v, zbuf, acc):
    c = jax.lax.axis_index("c")
    s = jax.lax.axis_index("s")
    wid = s * NC + c
    one16 = jnp.ones((16,), jnp.float32)
    zero16 = jnp.zeros((16,), jnp.float32)

    def _fill_ones(i, _):
        ones_v[pl.ds(i * 16, 16)] = one16
        return 0

    jax.lax.fori_loop(0, K // 16, _fill_ones, 0)

    def _fill_zero(i, _):
        zbuf[pl.ds(i * 16, 16)] = zero16
        return 0

    jax.lax.fori_loop(0, APT // 16, _fill_zero, 0)
    pltpu.sync_copy(zbuf, acc.at[pl.ds(s * APT, APT)])
    plsc.subcore_barrier()
    pltpu.sync_copy(dst_hbm.at[wid], dst_v)

    def _step(j, _):
        pltpu.sync_copy(ones_v, acc.at[dst_v.at[j]], add=True)
        return 0

    jax.lax.fori_loop(0, CH, _step, 0)
    plsc.subcore_barrier()
    pltpu.sync_copy(acc.at[pl.ds(s * APT, APT)],
                    out_hbm.at[c, pl.ds(s * APT, APT)])


HH = H // 2


NBUF = 4


def _sc_scatter_body(ypl_hbm, ypr_hbm, src_hbm, dst_hbm, out_hbm,
                     src_v, dst_v, rows, gsems, ssems, acc):
    c = jax.lax.axis_index("c")
    s = jax.lax.axis_index("s")
    wid = s * NC + c
    zero16 = jnp.zeros((16,), jnp.float32)

    def _zrow(r, _):
        for m in range(HH // 16):
            rows[0][r, pl.ds(m * 16, 16)] = zero16
        return 0

    pltpu.sync_copy(src_hbm.at[wid], src_v)
    pltpu.sync_copy(dst_hbm.at[wid], dst_v)
    for half, tab in ((0, ypl_hbm), (1, ypr_hbm)):
        jax.lax.fori_loop(0, K, _zrow, 0)
        for q in range(APT // K):
            pltpu.sync_copy(rows[0], acc.at[pl.ds(s * APT + q * K, K)])
        plsc.subcore_barrier()

        def _gather(j, b, tab=tab):
            return pltpu.async_copy(tab.at[src_v.at[j]], rows[b], gsems[b])

        def _scatter(j, b):
            return pltpu.async_copy(rows[b], acc.at[dst_v.at[j]], ssems[b],
                                    add=True)

        for b in range(NBUF):
            _gather(b, b)

        def _step(g, _, tab=tab):
            base = g * NBUF
            for pair in (0, 1):
                bs = (2 * pair, 2 * pair + 1)
                for b in bs:
                    j = base + b
                    pltpu.make_async_copy(tab.at[src_v.at[j]], rows[b],
                                          gsems[b]).wait()
                    _scatter(j, b)
                for b in bs:
                    j = base + b
                    pltpu.make_async_copy(rows[b], acc.at[dst_v.at[j]],
                                          ssems[b]).wait()

                    @pl.when(j + NBUF < CH)
                    def _(j=j, b=b):
                        _gather(j + NBUF, b)
            return 0

        jax.lax.fori_loop(0, CH // NBUF, _step, 0)
        plsc.subcore_barrier()
        pltpu.sync_copy(acc.at[pl.ds(s * APT, APT)],
                        out_hbm.at[c, half, pl.ds(s * APT, APT)])


def _sc_maxpool_body(h_hbm, b_hbm, out_hbm, hbuf, bseg, mx):
    c = jax.lax.axis_index("c")
    s = jax.lax.axis_index("s")
    wid = s * NC + c
    neg16 = jnp.full((16,), -jnp.inf, jnp.float32)

    def _init(r, _):
        for m in range(H // 16):
            mx[r, pl.ds(m * 16, 16)] = neg16
        return 0

    jax.lax.fori_loop(0, G + 8, _init, 0)
    pltpu.sync_copy(b_hbm.at[pl.ds(wid * RPT, RPT)], bseg)

    def _chunk(q, _):
        pltpu.sync_copy(h_hbm.at[pl.ds(wid * RPT + q * 64, 64)], hbuf)

        def _grp(gi, _):
            sv = bseg[pl.ds(q * 64 + gi * 16, 16)]
            for t in range(16):
                seg = sv[t]
                r = gi * 16 + t
                for m in range(H // 16):
                    sl = pl.ds(m * 16, 16)
                    mx[seg, sl] = jnp.maximum(mx[seg, sl], hbuf[r, sl])
            return 0

        jax.lax.fori_loop(0, 4, _grp, 0)
        return 0

    jax.lax.fori_loop(0, RPT // 64, _chunk, 0)
    pltpu.sync_copy(mx.at[pl.ds(0, G)], out_hbm.at[wid])


@functools.cache
def _sc_kernels():
    mesh = plsc.VectorSubcoreMesh(core_axis_name="c", subcore_axis_name="s",
                                  num_cores=NC, num_subcores=NS)
    deg = pl.kernel(
        _sc_deg_body,
        out_type=jax.ShapeDtypeStruct((NC, NPAD), jnp.float32),
        mesh=mesh,
        scratch_types=[
            pltpu.VMEM((CH, K), jnp.int32),
            pltpu.VMEM((K,), jnp.float32),
            pltpu.VMEM((APT,), jnp.float32),
            pltpu.VMEM_SHARED((NPAD,), jnp.float32),
        ],
    )
    scatter = pl.kernel(
        _sc_scatter_body,
        out_type=jax.ShapeDtypeStruct((NC, 2, NPAD, HH), jnp.float32),
        mesh=mesh,
        compiler_params=pltpu.CompilerParams(use_tc_tiling_on_sc=False),
        scratch_types=[
            pltpu.VMEM((CH, K), jnp.int32),
            pltpu.VMEM((CH, K), jnp.int32),
            [pltpu.VMEM((K, HH), jnp.float32) for _ in range(NBUF)],
            [pltpu.SemaphoreType.DMA for _ in range(NBUF)],
            [pltpu.SemaphoreType.DMA for _ in range(NBUF)],
            pltpu.VMEM_SHARED((NPAD, HH), jnp.float32),
        ],
    )
    maxpool = pl.kernel(
        _sc_maxpool_body,
        out_type=jax.ShapeDtypeStruct((NW, G, H), jnp.float32),
        mesh=mesh,
        scratch_types=[
            pltpu.VMEM((64, H), jnp.float32),
            pltpu.VMEM((RPT,), jnp.int32),
            pltpu.VMEM((G + 8, H), jnp.float32),
        ],
    )
    return deg, scatter, maxpool


def _sc_deg(dst3):
    return _sc_kernels()[0](dst3)


def _sc_scatter(ypl, ypr, src3, dst3):
    return _sc_kernels()[1](ypl, ypr, src3, dst3)


def _sc_maxpool(h, batchp):
    return _sc_kernels()[2](h, batchp)


BLK = 1024
GRID = NPAD // BLK


def _layer0_body(x_ref, d0_ref, d1_ref, w_ref, ypl_ref, ypr_ref, dinv_ref):
    dinv = jax.lax.rsqrt(1.0 + d0_ref[...] + d1_ref[...])
    dinv_ref[...] = dinv
    y = jnp.dot(x_ref[...], w_ref[...], preferred_element_type=jnp.float32)
    yp = y * dinv
    ypl_ref[...] = yp[:, :HH]
    ypr_ref[...] = yp[:, HH:]


_tc_layer0 = pl.pallas_call(
    _layer0_body,
    grid=(GRID,),
    in_specs=[
        pl.BlockSpec((BLK, D), lambda i: (i, 0)),
        pl.BlockSpec((BLK, 1), lambda i: (i, 0)),
        pl.BlockSpec((BLK, 1), lambda i: (i, 0)),
        pl.BlockSpec((D, H), lambda i: (0, 0)),
    ],
    out_specs=[
        pl.BlockSpec((BLK, HH), lambda i: (i, 0)),
        pl.BlockSpec((BLK, HH), lambda i: (i, 0)),
        pl.BlockSpec((BLK, 1), lambda i: (i, 0)),
    ],
    out_shape=[
        jax.ShapeDtypeStruct((NPAD, HH), jnp.float32),
        jax.ShapeDtypeStruct((NPAD, HH), jnp.float32),
        jax.ShapeDtypeStruct((NPAD, 1), jnp.float32),
    ],
)


def _state(p_ref, yppl_ref, yppr_ref, dinv_ref, b_ref, g_ref, bb_ref):
    accl = p_ref[0] + p_ref[2] + yppl_ref[...]
    accr = p_ref[1] + p_ref[3] + yppr_ref[...]
    agg = (dinv_ref[...] * jnp.concatenate([accl, accr], axis=1)
           + b_ref[...])
    return jnp.maximum(agg * BN_SCALE * g_ref[...] + bb_ref[...], 0.0)


def _layer_body(p_ref, yppl_ref, yppr_ref, dinv_ref, b_ref, g_ref, bb_ref,
                w_ref, ypl_ref, ypr_ref):
    st = _state(p_ref, yppl_ref, yppr_ref, dinv_ref, b_ref, g_ref, bb_ref)
    y = jnp.dot(st, w_ref[...], preferred_element_type=jnp.float32)
    yp = y * dinv_ref[...]
    ypl_ref[...] = yp[:, :HH]
    ypr_ref[...] = yp[:, HH:]


_tc_layer = pl.pallas_call(
    _layer_body,
    grid=(GRID,),
    in_specs=[
        pl.BlockSpec((4, BLK, HH), lambda i: (0, i, 0)),
        pl.BlockSpec((BLK, HH), lambda i: (i, 0)),
        pl.BlockSpec((BLK, HH), lambda i: (i, 0)),
        pl.BlockSpec((BLK, 1), lambda i: (i, 0)),
        pl.BlockSpec((1, H), lambda i: (0, 0)),
        pl.BlockSpec((1, H), lambda i: (0, 0)),
        pl.BlockSpec((1, H), lambda i: (0, 0)),
        pl.BlockSpec((H, H), lambda i: (0, 0)),
    ],
    out_specs=[
        pl.BlockSpec((BLK, HH), lambda i: (i, 0)),
        pl.BlockSpec((BLK, HH), lambda i: (i, 0)),
    ],
    out_shape=[
        jax.ShapeDtypeStruct((NPAD, HH), jnp.float32),
        jax.ShapeDtypeStruct((NPAD, HH), jnp.float32),
    ],
)


def _final_body(p_ref, yppl_ref, yppr_ref, dinv_ref, b_ref, g_ref, bb_ref,
                batch_ref, h_ref, ssum_ref, cnt_ref):
    i = pl.program_id(0)
    h = _state(p_ref, yppl_ref, yppr_ref, dinv_ref, b_ref, g_ref, bb_ref)
    h_ref[...] = h
    oh = (batch_ref[...] == jax.lax.broadcasted_iota(jnp.int32, (BLK, G), 1))
    oh = oh.astype(jnp.float32)
    dn = (((0,), (0,)), ((), ()))
    ps = jax.lax.dot_general(oh, h, dn, preferred_element_type=jnp.float32)
    pc = jax.lax.dot_general(oh, jnp.ones((BLK, H), jnp.float32), dn,
                             preferred_element_type=jnp.float32)

    @pl.when(i == 0)
    def _():
        ssum_ref[...] = ps
        cnt_ref[...] = pc

    @pl.when(i != 0)
    def _():
        ssum_ref[...] += ps
        cnt_ref[...] += pc


_tc_final = pl.pallas_call(
    _final_body,
    grid=(GRID,),
    in_specs=[
        pl.BlockSpec((4, BLK, HH), lambda i: (0, i, 0)),
        pl.BlockSpec((BLK, HH), lambda i: (i, 0)),
        pl.BlockSpec((BLK, HH), lambda i: (i, 0)),
        pl.BlockSpec((BLK, 1), lambda i: (i, 0)),
        pl.BlockSpec((1, H), lambda i: (0, 0)),
        pl.BlockSpec((1, H), lambda i: (0, 0)),
        pl.BlockSpec((1, H), lambda i: (0, 0)),
        pl.BlockSpec((BLK, 1), lambda i: (i, 0)),
    ],
    out_specs=[
        pl.BlockSpec((BLK, H), lambda i: (i, 0)),
        pl.BlockSpec((G, H), lambda i: (0, 0)),
        pl.BlockSpec((G, H), lambda i: (0, 0)),
    ],
    out_shape=[
        jax.ShapeDtypeStruct((NPAD, H), jnp.float32),
        jax.ShapeDtypeStruct((G, H), jnp.float32),
        jax.ShapeDtypeStruct((G, H), jnp.float32),
    ],
)


def _head_body(ssum_ref, cnt_ref, maxp_ref, gfin_ref,
               gw1, gb1, gg1, gbb1, gw2, gb2, gg2, gbb2,
               fw1, fb1, fg1, fbb1, fw2, fb2, fg2, fbb2, fw3, fb3,
               z_ref):
    cnt = cnt_ref[:, 0:1]
    ssum = ssum_ref[...]
    smax = jnp.max(maxp_ref[...], axis=0)
    x1 = ssum / jnp.maximum(cnt, 1.0)
    x2 = jnp.where(cnt > 0.0, smax, 0.0)

    def bnrelu(t, g, bb):
        return jnp.maximum(t * BN_SCALE * g[...] + bb[...], 0.0)

    gf = jnp.dot(gfin_ref[...], gw1[...],
                 preferred_element_type=jnp.float32) + gb1[...]
    gf = bnrelu(gf, gg1, gbb1)
    gf = jnp.dot(gf, gw2[...], preferred_element_type=jnp.float32) + gb2[...]
    gf = bnrelu(gf, gg2, gbb2)
    fused = jnp.concatenate([x1, x2, ssum, gf], axis=1)
    z = jnp.dot(fused, fw1[...], preferred_element_type=jnp.float32) + fb1[...]
    z = bnrelu(z, fg1, fbb1)
    z = jnp.dot(z, fw2[...], preferred_element_type=jnp.float32) + fb2[...]
    z = bnrelu(z, fg2, fbb2)
    z_ref[...] = jnp.dot(z, fw3[...],
                         preferred_element_type=jnp.float32) + fb3[...]


_tc_head = pl.pallas_call(
    _head_body,
    out_shape=jax.ShapeDtypeStruct((G, 1), jnp.float32),
)


def kernel(x, edge_index, batch, graph_features, params):
    src = edge_index[0].astype(jnp.int32)
    dst = edge_index[1].astype(jnp.int32)
    src3 = jnp.concatenate(
        [src, jnp.zeros((E_PAD - E,), jnp.int32)]).reshape(NW, CH, K)
    dst3 = jnp.concatenate(
        [dst, jnp.full((E_PAD - E,), N, jnp.int32)]).reshape(NW, CH, K)
    batchp = jnp.concatenate(
        [batch.astype(jnp.int32), jnp.full((NPAD - N,), G, jnp.int32)])
    xp = jnp.pad(x, ((0, NPAD - N), (0, 0)))

    def row(v):
        return v.reshape(1, -1)

    degp = _sc_deg(dst3)
    d0 = degp[0].reshape(NPAD, 1)
    d1 = degp[1].reshape(NPAD, 1)
    ypl, ypr, dinv = _tc_layer0(xp, d0, d1, params["gcn_w0"])
    for l in range(1, NLAYERS):
        p = _sc_scatter(ypl, ypr, src3, dst3).reshape(4, NPAD, HH)
        ypl, ypr = _tc_layer(p, ypl, ypr, dinv,
                             row(params[f"gcn_b{l-1}"]),
                             row(params[f"bn_g{l-1}"]),
                             row(params[f"bn_b{l-1}"]), params[f"gcn_w{l}"])
    p = _sc_scatter(ypl, ypr, src3, dst3).reshape(4, NPAD, HH)
    h, ssum, cnt2 = _tc_final(p, ypl, ypr, dinv,
                              row(params["gcn_b3"]), row(params["bn_g3"]),
                              row(params["bn_b3"]),
                              batchp.reshape(NPAD, 1))
    maxp = _sc_maxpool(h, batchp)
    z = _tc_head(ssum, cnt2, maxp, graph_features,
                 params["gm_w1"], row(params["gm_b1"]),
                 row(params["gm_g1"]), row(params["gm_bb1"]),
                 params["gm_w2"], row(params["gm_b2"]),
                 row(params["gm_g2"]), row(params["gm_bb2"]),
                 params["f_w1"], row(params["f_b1"]),
                 row(params["f_g1"]), row(params["f_bb1"]),
                 params["f_w2"], row(params["f_b2"]),
                 row(params["f_g2"]), row(params["f_bb2"]),
                 params["f_w3"], row(params["f_b3"]))
    return z


# speedup vs baseline: 6.4280x; 1.0011x over previous
import functools

import jax
import jax.numpy as jnp
from jax.experimental import pallas as pl
from jax.experimental.pallas import tpu as pltpu
from jax.experimental.pallas import tpu_sc as plsc

N = 10000
E = 320000
D = 128
H = 128
G = 64
GF = 32
NLAYERS = 4
BN_SCALE = (1.0 + 1e-5) ** -0.5

NC = 2
NS = 16
NW = NC * NS
K = 128
CH = 80
E_PAD = NW * CH * K
NPAD = 10240
APT = NPAD // NS
RPT = NPAD // NW


def _sc_deg_body(dst_hbm, out_hbm, dst_v, ones_v, zbuf, acc):
    c = jax.lax.axis_index("c")
    s = jax.lax.axis_index("s")
    wid = s * NC + c
    one16 = jnp.ones((16,), jnp.float32)
    zero16 = jnp.zeros((16,), jnp.float32)

    def _fill_ones(i, _):
        ones_v[pl.ds(i * 16, 16)] = one16
        return 0

    jax.lax.fori_loop(0, K // 16, _fill_ones, 0)

    def _fill_zero(i, _):
        zbuf[pl.ds(i * 16, 16)] = zero16
        return 0

    jax.lax.fori_loop(0, APT // 16, _fill_zero, 0)
    pltpu.sync_copy(zbuf, acc.at[pl.ds(s * APT, APT)])
    plsc.subcore_barrier()
    pltpu.sync_copy(dst_hbm.at[wid], dst_v)

    def _step(j, _):
        pltpu.sync_copy(ones_v, acc.at[dst_v.at[j]], add=True)
        return 0

    jax.lax.fori_loop(0, CH, _step, 0)
    plsc.subcore_barrier()
    pltpu.sync_copy(acc.at[pl.ds(s * APT, APT)],
                    out_hbm.at[c, pl.ds(s * APT, APT)])


HH = H // 2


NBUF = 8


def _sc_scatter_body(ypl_hbm, ypr_hbm, src_hbm, dst_hbm, out_hbm,
                     src_v, dst_v, rows, gsems, ssems, acc):
    c = jax.lax.axis_index("c")
    s = jax.lax.axis_index("s")
    wid = s * NC + c
    zero16 = jnp.zeros((16,), jnp.float32)

    def _zrow(r, _):
        for m in range(HH // 16):
            rows[0][r, pl.ds(m * 16, 16)] = zero16
        return 0

    pltpu.sync_copy(src_hbm.at[wid], src_v)
    pltpu.sync_copy(dst_hbm.at[wid], dst_v)
    for half, tab in ((0, ypl_hbm), (1, ypr_hbm)):
        jax.lax.fori_loop(0, K, _zrow, 0)
        for q in range(APT // K):
            pltpu.sync_copy(rows[0], acc.at[pl.ds(s * APT + q * K, K)])
        plsc.subcore_barrier()

        def _gather(j, b, tab=tab):
            return pltpu.async_copy(tab.at[src_v.at[j]], rows[b], gsems[b])

        def _scatter(j, b):
            return pltpu.async_copy(rows[b], acc.at[dst_v.at[j]], ssems[b],
                                    add=True)

        for b in range(NBUF):
            _gather(b, b)

        def _step(g, _, tab=tab):
            base = g * NBUF
            for pair in range(NBUF // 2):
                bs = (2 * pair, 2 * pair + 1)
                for b in bs:
                    j = base + b
                    pltpu.make_async_copy(tab.at[src_v.at[j]], rows[b],
                                          gsems[b]).wait()
                    _scatter(j, b)
                for b in bs:
                    j = base + b
                    pltpu.make_async_copy(rows[b], acc.at[dst_v.at[j]],
                                          ssems[b]).wait()

                    @pl.when(j + NBUF < CH)
                    def _(j=j, b=b):
                        _gather(j + NBUF, b)
            return 0

        jax.lax.fori_loop(0, CH // NBUF, _step, 0)
        plsc.subcore_barrier()
        pltpu.sync_copy(acc.at[pl.ds(s * APT, APT)],
                        out_hbm.at[c, half, pl.ds(s * APT, APT)])


def _sc_maxpool_body(h_hbm, b_hbm, out_hbm, hbuf, bseg, mx):
    c = jax.lax.axis_index("c")
    s = jax.lax.axis_index("s")
    wid = s * NC + c
    neg16 = jnp.full((16,), -jnp.inf, jnp.float32)

    def _init(r, _):
        for m in range(H // 16):
            mx[r, pl.ds(m * 16, 16)] = neg16
        return 0

    jax.lax.fori_loop(0, G + 8, _init, 0)
    pltpu.sync_copy(b_hbm.at[pl.ds(wid * RPT, RPT)], bseg)

    def _chunk(q, _):
        pltpu.sync_copy(h_hbm.at[pl.ds(wid * RPT + q * 64, 64)], hbuf)

        def _grp(gi, _):
            sv = bseg[pl.ds(q * 64 + gi * 16, 16)]
            for t in range(16):
                seg = sv[t]
                r = gi * 16 + t
                for m in range(H // 16):
                    sl = pl.ds(m * 16, 16)
                    mx[seg, sl] = jnp.maximum(mx[seg, sl], hbuf[r, sl])
            return 0

        jax.lax.fori_loop(0, 4, _grp, 0)
        return 0

    jax.lax.fori_loop(0, RPT // 64, _chunk, 0)
    pltpu.sync_copy(mx.at[pl.ds(0, G)], out_hbm.at[wid])


@functools.cache
def _sc_kernels():
    mesh = plsc.VectorSubcoreMesh(core_axis_name="c", subcore_axis_name="s",
                                  num_cores=NC, num_subcores=NS)
    deg = pl.kernel(
        _sc_deg_body,
        out_type=jax.ShapeDtypeStruct((NC, NPAD), jnp.float32),
        mesh=mesh,
        scratch_types=[
            pltpu.VMEM((CH, K), jnp.int32),
            pltpu.VMEM((K,), jnp.float32),
            pltpu.VMEM((APT,), jnp.float32),
            pltpu.VMEM_SHARED((NPAD,), jnp.float32),
        ],
    )
    scatter = pl.kernel(
        _sc_scatter_body,
        out_type=jax.ShapeDtypeStruct((NC, 2, NPAD, HH), jnp.float32),
        mesh=mesh,
        compiler_params=pltpu.CompilerParams(use_tc_tiling_on_sc=False),
        scratch_types=[
            pltpu.VMEM((CH, K), jnp.int32),
            pltpu.VMEM((CH, K), jnp.int32),
            [pltpu.VMEM((K, HH), jnp.float32) for _ in range(NBUF)],
            [pltpu.SemaphoreType.DMA for _ in range(NBUF)],
            [pltpu.SemaphoreType.DMA for _ in range(NBUF)],
            pltpu.VMEM_SHARED((NPAD, HH), jnp.float32),
        ],
    )
    maxpool = pl.kernel(
        _sc_maxpool_body,
        out_type=jax.ShapeDtypeStruct((NW, G, H), jnp.float32),
        mesh=mesh,
        scratch_types=[
            pltpu.VMEM((64, H), jnp.float32),
            pltpu.VMEM((RPT,), jnp.int32),
            pltpu.VMEM((G + 8, H), jnp.float32),
        ],
    )
    return deg, scatter, maxpool


def _sc_deg(dst3):
    return _sc_kernels()[0](dst3)


def _sc_scatter(ypl, ypr, src3, dst3):
    return _sc_kernels()[1](ypl, ypr, src3, dst3)


def _sc_maxpool(h, batchp):
    return _sc_kernels()[2](h, batchp)


BLK = 1024
GRID = NPAD // BLK


def _layer0_body(x_ref, d0_ref, d1_ref, w_ref, ypl_ref, ypr_ref, dinv_ref):
    dinv = jax.lax.rsqrt(1.0 + d0_ref[...] + d1_ref[...])
    dinv_ref[...] = dinv
    y = jnp.dot(x_ref[...], w_ref[...], preferred_element_type=jnp.float32)
    yp = y * dinv
    ypl_ref[...] = yp[:, :HH]
    ypr_ref[...] = yp[:, HH:]


_tc_layer0 = pl.pallas_call(
    _layer0_body,
    grid=(GRID,),
    in_specs=[
        pl.BlockSpec((BLK, D), lambda i: (i, 0)),
        pl.BlockSpec((BLK, 1), lambda i: (i, 0)),
        pl.BlockSpec((BLK, 1), lambda i: (i, 0)),
        pl.BlockSpec((D, H), lambda i: (0, 0)),
    ],
    out_specs=[
        pl.BlockSpec((BLK, HH), lambda i: (i, 0)),
        pl.BlockSpec((BLK, HH), lambda i: (i, 0)),
        pl.BlockSpec((BLK, 1), lambda i: (i, 0)),
    ],
    out_shape=[
        jax.ShapeDtypeStruct((NPAD, HH), jnp.float32),
        jax.ShapeDtypeStruct((NPAD, HH), jnp.float32),
        jax.ShapeDtypeStruct((NPAD, 1), jnp.float32),
    ],
)


def _state(p_ref, yppl_ref, yppr_ref, dinv_ref, b_ref, g_ref, bb_ref):
    accl = p_ref[0] + p_ref[2] + yppl_ref[...]
    accr = p_ref[1] + p_ref[3] + yppr_ref[...]
    agg = (dinv_ref[...] * jnp.concatenate([accl, accr], axis=1)
           + b_ref[...])
    return jnp.maximum(agg * BN_SCALE * g_ref[...] + bb_ref[...], 0.0)


def _layer_body(p_ref, yppl_ref, yppr_ref, dinv_ref, b_ref, g_ref, bb_ref,
                w_ref, ypl_ref, ypr_ref):
    st = _state(p_ref, yppl_ref, yppr_ref, dinv_ref, b_ref, g_ref, bb_ref)
    y = jnp.dot(st, w_ref[...], preferred_element_type=jnp.float32)
    yp = y * dinv_ref[...]
    ypl_ref[...] = yp[:, :HH]
    ypr_ref[...] = yp[:, HH:]


_tc_layer = pl.pallas_call(
    _layer_body,
    grid=(GRID,),
    in_specs=[
        pl.BlockSpec((4, BLK, HH), lambda i: (0, i, 0)),
        pl.BlockSpec((BLK, HH), lambda i: (i, 0)),
        pl.BlockSpec((BLK, HH), lambda i: (i, 0)),
        pl.BlockSpec((BLK, 1), lambda i: (i, 0)),
        pl.BlockSpec((1, H), lambda i: (0, 0)),
        pl.BlockSpec((1, H), lambda i: (0, 0)),
        pl.BlockSpec((1, H), lambda i: (0, 0)),
        pl.BlockSpec((H, H), lambda i: (0, 0)),
    ],
    out_specs=[
        pl.BlockSpec((BLK, HH), lambda i: (i, 0)),
        pl.BlockSpec((BLK, HH), lambda i: (i, 0)),
    ],
    out_shape=[
        jax.ShapeDtypeStruct((NPAD, HH), jnp.float32),
        jax.ShapeDtypeStruct((NPAD, HH), jnp.float32),
    ],
)


def _final_body(p_ref, yppl_ref, yppr_ref, dinv_ref, b_ref, g_ref, bb_ref,
                batch_ref, h_ref, ssum_ref, cnt_ref):
    i = pl.program_id(0)
    h = _state(p_ref, yppl_ref, yppr_ref, dinv_ref, b_ref, g_ref, bb_ref)
    h_ref[...] = h
    oh = (batch_ref[...] == jax.lax.broadcasted_iota(jnp.int32, (BLK, G), 1))
    oh = oh.astype(jnp.float32)
    dn = (((0,), (0,)), ((), ()))
    ps = jax.lax.dot_general(oh, h, dn, preferred_element_type=jnp.float32)
    pc = jax.lax.dot_general(oh, jnp.ones((BLK, H), jnp.float32), dn,
                             preferred_element_type=jnp.float32)

    @pl.when(i == 0)
    def _():
        ssum_ref[...] = ps
        cnt_ref[...] = pc

    @pl.when(i != 0)
    def _():
        ssum_ref[...] += ps
        cnt_ref[...] += pc


_tc_final = pl.pallas_call(
    _final_body,
    grid=(GRID,),
    in_specs=[
        pl.BlockSpec((4, BLK, HH), lambda i: (0, i, 0)),
        pl.BlockSpec((BLK, HH), lambda i: (i, 0)),
        pl.BlockSpec((BLK, HH), lambda i: (i, 0)),
        pl.BlockSpec((BLK, 1), lambda i: (i, 0)),
        pl.BlockSpec((1, H), lambda i: (0, 0)),
        pl.BlockSpec((1, H), lambda i: (0, 0)),
        pl.BlockSpec((1, H), lambda i: (0, 0)),
        pl.BlockSpec((BLK, 1), lambda i: (i, 0)),
    ],
    out_specs=[
        pl.BlockSpec((BLK, H), lambda i: (i, 0)),
        pl.BlockSpec((G, H), lambda i: (0, 0)),
        pl.BlockSpec((G, H), lambda i: (0, 0)),
    ],
    out_shape=[
        jax.ShapeDtypeStruct((NPAD, H), jnp.float32),
        jax.ShapeDtypeStruct((G, H), jnp.float32),
        jax.ShapeDtypeStruct((G, H), jnp.float32),
    ],
)


def _head_body(ssum_ref, cnt_ref, maxp_ref, gfin_ref,
               gw1, gb1, gg1, gbb1, gw2, gb2, gg2, gbb2,
               fw1, fb1, fg1, fbb1, fw2, fb2, fg2, fbb2, fw3, fb3,
               z_ref):
    cnt = cnt_ref[:, 0:1]
    ssum = ssum_ref[...]
    smax = jnp.max(maxp_ref[...], axis=0)
    x1 = ssum / jnp.maximum(cnt, 1.0)
    x2 = jnp.where(cnt > 0.0, smax, 0.0)

    def bnrelu(t, g, bb):
        return jnp.maximum(t * BN_SCALE * g[...] + bb[...], 0.0)

    gf = jnp.dot(gfin_ref[...], gw1[...],
                 preferred_element_type=jnp.float32) + gb1[...]
    gf = bnrelu(gf, gg1, gbb1)
    gf = jnp.dot(gf, gw2[...], preferred_element_type=jnp.float32) + gb2[...]
    gf = bnrelu(gf, gg2, gbb2)
    fused = jnp.concatenate([x1, x2, ssum, gf], axis=1)
    z = jnp.dot(fused, fw1[...], preferred_element_type=jnp.float32) + fb1[...]
    z = bnrelu(z, fg1, fbb1)
    z = jnp.dot(z, fw2[...], preferred_element_type=jnp.float32) + fb2[...]
    z = bnrelu(z, fg2, fbb2)
    z_ref[...] = jnp.dot(z, fw3[...],
                         preferred_element_type=jnp.float32) + fb3[...]


_tc_head = pl.pallas_call(
    _head_body,
    out_shape=jax.ShapeDtypeStruct((G, 1), jnp.float32),
)


def kernel(x, edge_index, batch, graph_features, params):
    src = edge_index[0].astype(jnp.int32)
    dst = edge_index[1].astype(jnp.int32)
    src3 = jnp.concatenate(
        [src, jnp.zeros((E_PAD - E,), jnp.int32)]).reshape(NW, CH, K)
    dst3 = jnp.concatenate(
        [dst, jnp.full((E_PAD - E,), N, jnp.int32)]).reshape(NW, CH, K)
    batchp = jnp.concatenate(
        [batch.astype(jnp.int32), jnp.full((NPAD - N,), G, jnp.int32)])
    xp = jnp.pad(x, ((0, NPAD - N), (0, 0)))

    def row(v):
        return v.reshape(1, -1)

    degp = _sc_deg(dst3)
    d0 = degp[0].reshape(NPAD, 1)
    d1 = degp[1].reshape(NPAD, 1)
    ypl, ypr, dinv = _tc_layer0(xp, d0, d1, params["gcn_w0"])
    for l in range(1, NLAYERS):
        p = _sc_scatter(ypl, ypr, src3, dst3).reshape(4, NPAD, HH)
        ypl, ypr = _tc_layer(p, ypl, ypr, dinv,
                             row(params[f"gcn_b{l-1}"]),
                             row(params[f"bn_g{l-1}"]),
                             row(params[f"bn_b{l-1}"]), params[f"gcn_w{l}"])
    p = _sc_scatter(ypl, ypr, src3, dst3).reshape(4, NPAD, HH)
    h, ssum, cnt2 = _tc_final(p, ypl, ypr, dinv,
                              row(params["gcn_b3"]), row(params["bn_g3"]),
                              row(params["bn_b3"]),
                              batchp.reshape(NPAD, 1))
    maxp = _sc_maxpool(h, batchp)
    z = _tc_head(ssum, cnt2, maxp, graph_features,
                 params["gm_w1"], row(params["gm_b1"]),
                 row(params["gm_g1"]), row(params["gm_bb1"]),
                 params["gm_w2"], row(params["gm_b2"]),
                 row(params["gm_g2"]), row(params["gm_bb2"]),
                 params["f_w1"], row(params["f_b1"]),
                 row(params["f_g1"]), row(params["f_bb1"]),
                 params["f_w2"], row(params["f_b2"]),
                 row(params["f_g2"]), row(params["f_bb2"]),
                 params["f_w3"], row(params["f_b3"]))
    return z


# speedup vs baseline: 6.4303x; 1.0004x over previous
import functools

import jax
import jax.numpy as jnp
from jax.experimental import pallas as pl
from jax.experimental.pallas import tpu as pltpu
from jax.experimental.pallas import tpu_sc as plsc

N = 10000
E = 320000
D = 128
H = 128
G = 64
GF = 32
NLAYERS = 4
BN_SCALE = (1.0 + 1e-5) ** -0.5

NC = 2
NS = 16
NW = NC * NS
K = 128
CH = 80
E_PAD = NW * CH * K
NPAD = 10240
APT = NPAD // NS
RPT = NPAD // NW


def _sc_deg_body(dst_hbm, out_hbm, dst_v, ones_v, zbuf, acc):
    c = jax.lax.axis_index("c")
    s = jax.lax.axis_index("s")
    wid = s * NC + c
    one16 = jnp.ones((16,), jnp.float32)
    zero16 = jnp.zeros((16,), jnp.float32)

    def _fill_ones(i, _):
        ones_v[pl.ds(i * 16, 16)] = one16
        return 0

    jax.lax.fori_loop(0, K // 16, _fill_ones, 0)

    def _fill_zero(i, _):
        zbuf[pl.ds(i * 16, 16)] = zero16
        return 0

    jax.lax.fori_loop(0, APT // 16, _fill_zero, 0)
    pltpu.sync_copy(zbuf, acc.at[pl.ds(s * APT, APT)])
    plsc.subcore_barrier()
    pltpu.sync_copy(dst_hbm.at[wid], dst_v)

    def _step(j, _):
        pltpu.sync_copy(ones_v, acc.at[dst_v.at[j]], add=True)
        return 0

    jax.lax.fori_loop(0, R0, _step, 0)
    plsc.subcore_barrier()
    pltpu.sync_copy(acc.at[pl.ds(s * APT, APT)],
                    out_hbm.at[c, pl.ds(s * APT, APT)])


HH = H // 2


NBUF = 4
R0 = 128
R1 = 32
NSLAB = NW


def _sc_scatter_body(ypl_hbm, ypr_hbm, src_hbm, dst_hbm, out_hbm,
                     src_v, dst_v, rows, gsems, ssems, acc):
    c = jax.lax.axis_index("c")
    s = jax.lax.axis_index("s")
    wid = c * NS + s
    zero16 = jnp.zeros((16,), jnp.float32)
    rc = jnp.where(c == 0, R0, R1)
    nsteps = jnp.where(c == 0, R0 // NBUF, R1 // NBUF)

    def _zrow(r, _):
        for m in range(HH // 16):
            rows[0][r, pl.ds(m * 16, 16)] = zero16
        return 0

    pltpu.sync_copy(src_hbm.at[wid], src_v)
    pltpu.sync_copy(dst_hbm.at[wid], dst_v)
    for half, tab in ((0, ypl_hbm), (1, ypr_hbm)):
        jax.lax.fori_loop(0, K, _zrow, 0)
        for q in range(APT // K):
            pltpu.sync_copy(rows[0], acc.at[pl.ds(s * APT + q * K, K)])
        plsc.subcore_barrier()

        def _gather(j, b, tab=tab):
            return pltpu.async_copy(tab.at[src_v.at[j]], rows[b], gsems[b])

        def _scatter(j, b):
            return pltpu.async_copy(rows[b], acc.at[dst_v.at[j]], ssems[b],
                                    add=True)

        for b in range(NBUF):
            _gather(b, b)

        def _step(g, _, tab=tab):
            base = g * NBUF
            for pair in range(NBUF // 2):
                bs = (2 * pair, 2 * pair + 1)
                for b in bs:
                    j = base + b
                    pltpu.make_async_copy(tab.at[src_v.at[j]], rows[b],
                                          gsems[b]).wait()
                    _scatter(j, b)
                for b in bs:
                    j = base + b
                    pltpu.make_async_copy(rows[b], acc.at[dst_v.at[j]],
                                          ssems[b]).wait()

                    @pl.when(j + NBUF < rc)
                    def _(j=j, b=b):
                        _gather(j + NBUF, b)
            return 0

        jax.lax.fori_loop(0, nsteps, _step, 0)
        plsc.subcore_barrier()
        pltpu.sync_copy(acc.at[pl.ds(s * APT, APT)],
                        out_hbm.at[c, half, pl.ds(s * APT, APT)])


def _sc_maxpool_body(h_hbm, b_hbm, out_hbm, hbuf, bseg, mx):
    c = jax.lax.axis_index("c")
    s = jax.lax.axis_index("s")
    wid = s * NC + c
    neg16 = jnp.full((16,), -jnp.inf, jnp.float32)

    def _init(r, _):
        for m in range(H // 16):
            mx[r, pl.ds(m * 16, 16)] = neg16
        return 0

    jax.lax.fori_loop(0, G + 8, _init, 0)
    pltpu.sync_copy(b_hbm.at[pl.ds(wid * RPT, RPT)], bseg)

    def _chunk(q, _):
        pltpu.sync_copy(h_hbm.at[pl.ds(wid * RPT + q * 64, 64)], hbuf)

        def _grp(gi, _):
            sv = bseg[pl.ds(q * 64 + gi * 16, 16)]
            for t in range(16):
                seg = sv[t]
                r = gi * 16 + t
                for m in range(H // 16):
                    sl = pl.ds(m * 16, 16)
                    mx[seg, sl] = jnp.maximum(mx[seg, sl], hbuf[r, sl])
            return 0

        jax.lax.fori_loop(0, 4, _grp, 0)
        return 0

    jax.lax.fori_loop(0, RPT // 64, _chunk, 0)
    pltpu.sync_copy(mx.at[pl.ds(0, G)], out_hbm.at[wid])


@functools.cache
def _sc_kernels():
    mesh = plsc.VectorSubcoreMesh(core_axis_name="c", subcore_axis_name="s",
                                  num_cores=NC, num_subcores=NS)
    deg = pl.kernel(
        _sc_deg_body,
        out_type=jax.ShapeDtypeStruct((NC, NPAD), jnp.float32),
        mesh=mesh,
        scratch_types=[
            pltpu.VMEM((R0, K), jnp.int32),
            pltpu.VMEM((K,), jnp.float32),
            pltpu.VMEM((APT,), jnp.float32),
            pltpu.VMEM_SHARED((NPAD,), jnp.float32),
        ],
    )
    scatter = pl.kernel(
        _sc_scatter_body,
        out_type=jax.ShapeDtypeStruct((NC, 2, NPAD, HH), jnp.float32),
        mesh=mesh,
        compiler_params=pltpu.CompilerParams(use_tc_tiling_on_sc=False),
        scratch_types=[
            pltpu.VMEM((R0, K), jnp.int32),
            pltpu.VMEM((R0, K), jnp.int32),
            [pltpu.VMEM((K, HH), jnp.float32) for _ in range(NBUF)],
            [pltpu.SemaphoreType.DMA for _ in range(NBUF)],
            [pltpu.SemaphoreType.DMA for _ in range(NBUF)],
            pltpu.VMEM_SHARED((NPAD, HH), jnp.float32),
        ],
    )
    maxpool = pl.kernel(
        _sc_maxpool_body,
        out_type=jax.ShapeDtypeStruct((NW, G, H), jnp.float32),
        mesh=mesh,
        scratch_types=[
            pltpu.VMEM((64, H), jnp.float32),
            pltpu.VMEM((RPT,), jnp.int32),
            pltpu.VMEM((G + 8, H), jnp.float32),
        ],
    )
    return deg, scatter, maxpool


def _sc_deg(dst3):
    return _sc_kernels()[0](dst3)


def _sc_scatter(ypl, ypr, src3, dst3):
    return _sc_kernels()[1](ypl, ypr, src3, dst3)


def _sc_maxpool(h, batchp):
    return _sc_kernels()[2](h, batchp)


BLK = 1024
GRID = NPAD // BLK


def _layer0_body(x_ref, d0_ref, d1_ref, w_ref, ypl_ref, ypr_ref, dinv_ref):
    dinv = jax.lax.rsqrt(1.0 + d0_ref[...] + d1_ref[...])
    dinv_ref[...] = dinv
    y = jnp.dot(x_ref[...], w_ref[...], preferred_element_type=jnp.float32)
    yp = y * dinv
    ypl_ref[...] = yp[:, :HH]
    ypr_ref[...] = yp[:, HH:]


_tc_layer0 = pl.pallas_call(
    _layer0_body,
    grid=(GRID,),
    in_specs=[
        pl.BlockSpec((BLK, D), lambda i: (i, 0)),
        pl.BlockSpec((BLK, 1), lambda i: (i, 0)),
        pl.BlockSpec((BLK, 1), lambda i: (i, 0)),
        pl.BlockSpec((D, H), lambda i: (0, 0)),
    ],
    out_specs=[
        pl.BlockSpec((BLK, HH), lambda i: (i, 0)),
        pl.BlockSpec((BLK, HH), lambda i: (i, 0)),
        pl.BlockSpec((BLK, 1), lambda i: (i, 0)),
    ],
    out_shape=[
        jax.ShapeDtypeStruct((NPAD, HH), jnp.float32),
        jax.ShapeDtypeStruct((NPAD, HH), jnp.float32),
        jax.ShapeDtypeStruct((NPAD, 1), jnp.float32),
    ],
)


def _state(p_ref, yppl_ref, yppr_ref, dinv_ref, b_ref, g_ref, bb_ref):
    accl = p_ref[0] + p_ref[2] + yppl_ref[...]
    accr = p_ref[1] + p_ref[3] + yppr_ref[...]
    agg = (dinv_ref[...] * jnp.concatenate([accl, accr], axis=1)
           + b_ref[...])
    return jnp.maximum(agg * BN_SCALE * g_ref[...] + bb_ref[...], 0.0)


def _layer_body(p_ref, yppl_ref, yppr_ref, dinv_ref, b_ref, g_ref, bb_ref,
                w_ref, ypl_ref, ypr_ref):
    st = _state(p_ref, yppl_ref, yppr_ref, dinv_ref, b_ref, g_ref, bb_ref)
    y = jnp.dot(st, w_ref[...], preferred_element_type=jnp.float32)
    yp = y * dinv_ref[...]
    ypl_ref[...] = yp[:, :HH]
    ypr_ref[...] = yp[:, HH:]


_tc_layer = pl.pallas_call(
    _layer_body,
    grid=(GRID,),
    in_specs=[
        pl.BlockSpec((4, BLK, HH), lambda i: (0, i, 0)),
        pl.BlockSpec((BLK, HH), lambda i: (i, 0)),
        pl.BlockSpec((BLK, HH), lambda i: (i, 0)),
        pl.BlockSpec((BLK, 1), lambda i: (i, 0)),
        pl.BlockSpec((1, H), lambda i: (0, 0)),
        pl.BlockSpec((1, H), lambda i: (0, 0)),
        pl.BlockSpec((1, H), lambda i: (0, 0)),
        pl.BlockSpec((H, H), lambda i: (0, 0)),
    ],
    out_specs=[
        pl.BlockSpec((BLK, HH), lambda i: (i, 0)),
        pl.BlockSpec((BLK, HH), lambda i: (i, 0)),
    ],
    out_shape=[
        jax.ShapeDtypeStruct((NPAD, HH), jnp.float32),
        jax.ShapeDtypeStruct((NPAD, HH), jnp.float32),
    ],
)


def _final_body(p_ref, yppl_ref, yppr_ref, dinv_ref, b_ref, g_ref, bb_ref,
                batch_ref, h_ref, ssum_ref, cnt_ref):
    i = pl.program_id(0)
    h = _state(p_ref, yppl_ref, yppr_ref, dinv_ref, b_ref, g_ref, bb_ref)
    h_ref[...] = h
    oh = (batch_ref[...] == jax.lax.broadcasted_iota(jnp.int32, (BLK, G), 1))
    oh = oh.astype(jnp.float32)
    dn = (((0,), (0,)), ((), ()))
    ps = jax.lax.dot_general(oh, h, dn, preferred_element_type=jnp.float32)
    pc = jax.lax.dot_general(oh, jnp.ones((BLK, H), jnp.float32), dn,
                             preferred_element_type=jnp.float32)

    @pl.when(i == 0)
    def _():
        ssum_ref[...] = ps
        cnt_ref[...] = pc

    @pl.when(i != 0)
    def _():
        ssum_ref[...] += ps
        cnt_ref[...] += pc


_tc_final = pl.pallas_call(
    _final_body,
    grid=(GRID,),
    in_specs=[
        pl.BlockSpec((4, BLK, HH), lambda i: (0, i, 0)),
        pl.BlockSpec((BLK, HH), lambda i: (i, 0)),
        pl.BlockSpec((BLK, HH), lambda i: (i, 0)),
        pl.BlockSpec((BLK, 1), lambda i: (i, 0)),
        pl.BlockSpec((1, H), lambda i: (0, 0)),
        pl.BlockSpec((1, H), lambda i: (0, 0)),
        pl.BlockSpec((1, H), lambda i: (0, 0)),
        pl.BlockSpec((BLK, 1), lambda i: (i, 0)),
    ],
    out_specs=[
        pl.BlockSpec((BLK, H), lambda i: (i, 0)),
        pl.BlockSpec((G, H), lambda i: (0, 0)),
        pl.BlockSpec((G, H), lambda i: (0, 0)),
    ],
    out_shape=[
        jax.ShapeDtypeStruct((NPAD, H), jnp.float32),
        jax.ShapeDtypeStruct((G, H), jnp.float32),
        jax.ShapeDtypeStruct((G, H), jnp.float32),
    ],
)


def _head_body(ssum_ref, cnt_ref, maxp_ref, gfin_ref,
               gw1, gb1, gg1, gbb1, gw2, gb2, gg2, gbb2,
               fw1, fb1, fg1, fbb1, fw2, fb2, fg2, fbb2, fw3, fb3,
               z_ref):
    cnt = cnt_ref[:, 0:1]
    ssum = ssum_ref[...]
    smax = jnp.max(maxp_ref[...], axis=0)
    x1 = ssum / jnp.maximum(cnt, 1.0)
    x2 = jnp.where(cnt > 0.0, smax, 0.0)

    def bnrelu(t, g, bb):
        return jnp.maximum(t * BN_SCALE * g[...] + bb[...], 0.0)

    gf = jnp.dot(gfin_ref[...], gw1[...],
                 preferred_element_type=jnp.float32) + gb1[...]
    gf = bnrelu(gf, gg1, gbb1)
    gf = jnp.dot(gf, gw2[...], preferred_element_type=jnp.float32) + gb2[...]
    gf = bnrelu(gf, gg2, gbb2)
    fused = jnp.concatenate([x1, x2, ssum, gf], axis=1)
    z = jnp.dot(fused, fw1[...], preferred_element_type=jnp.float32) + fb1[...]
    z = bnrelu(z, fg1, fbb1)
    z = jnp.dot(z, fw2[...], preferred_element_type=jnp.float32) + fb2[...]
    z = bnrelu(z, fg2, fbb2)
    z_ref[...] = jnp.dot(z, fw3[...],
                         preferred_element_type=jnp.float32) + fb3[...]


_tc_head = pl.pallas_call(
    _head_body,
    out_shape=jax.ShapeDtypeStruct((G, 1), jnp.float32),
)


def kernel(x, edge_index, batch, graph_features, params):
    src = edge_index[0].astype(jnp.int32)
    dst = edge_index[1].astype(jnp.int32)
    e0 = NS * R0 * K
    e1r = NS * R1 * K
    def slabs(v, fill):
        p0 = v[:e0].reshape(NS, R0, K)
        p1 = jnp.concatenate(
            [v[e0:], jnp.full((e1r - (E - e0),), fill, jnp.int32)]
        ).reshape(NS, R1, K)
        p1 = jnp.concatenate(
            [p1, jnp.full((NS, R0 - R1, K), fill, jnp.int32)], axis=1)
        return jnp.concatenate([p0.reshape(NS, R0, K), p1], axis=0)
    src3 = slabs(src, 0)
    dst3 = slabs(dst, N)
    batchp = jnp.concatenate(
        [batch.astype(jnp.int32), jnp.full((NPAD - N,), G, jnp.int32)])
    xp = jnp.pad(x, ((0, NPAD - N), (0, 0)))

    def row(v):
        return v.reshape(1, -1)

    degp = _sc_deg(dst3)
    d0 = degp[0].reshape(NPAD, 1)
    d1 = degp[1].reshape(NPAD, 1)
    ypl, ypr, dinv = _tc_layer0(xp, d0, d1, params["gcn_w0"])
    for l in range(1, NLAYERS):
        p = _sc_scatter(ypl, ypr, src3, dst3).reshape(4, NPAD, HH)
        ypl, ypr = _tc_layer(p, ypl, ypr, dinv,
                             row(params[f"gcn_b{l-1}"]),
                             row(params[f"bn_g{l-1}"]),
                             row(params[f"bn_b{l-1}"]), params[f"gcn_w{l}"])
    p = _sc_scatter(ypl, ypr, src3, dst3).reshape(4, NPAD, HH)
    h, ssum, cnt2 = _tc_final(p, ypl, ypr, dinv,
                              row(params["gcn_b3"]), row(params["bn_g3"]),
                              row(params["bn_b3"]),
                              batchp.reshape(NPAD, 1))
    maxp = _sc_maxpool(h, batchp)
    z = _tc_head(ssum, cnt2, maxp, graph_features,
                 params["gm_w1"], row(params["gm_b1"]),
                 row(params["gm_g1"]), row(params["gm_bb1"]),
                 params["gm_w2"], row(params["gm_b2"]),
                 row(params["gm_g2"]), row(params["gm_bb2"]),
                 params["f_w1"], row(params["f_b1"]),
                 row(params["f_g1"]), row(params["f_bb1"]),
                 params["f_w2"], row(params["f_b2"]),
                 row(params["f_g2"]), row(params["f_bb2"]),
                 params["f_w3"], row(params["f_b3"]))
    return z


# speedup vs baseline: 6.7327x; 1.0470x over previous
import functools

import jax
import jax.numpy as jnp
from jax.experimental import pallas as pl
from jax.experimental.pallas import tpu as pltpu
from jax.experimental.pallas import tpu_sc as plsc

N = 10000
E = 320000
D = 128
H = 128
G = 64
GF = 32
NLAYERS = 4
BN_SCALE = (1.0 + 1e-5) ** -0.5

NC = 2
NS = 16
NW = NC * NS
K = 128
CH = 80
E_PAD = NW * CH * K
NPAD = 10240
APT = NPAD // NS
RPT = NPAD // NW


def _sc_deg_body(dst_hbm, out_hbm, dst_v, ones_v, zbuf, acc):
    c = jax.lax.axis_index("c")
    s = jax.lax.axis_index("s")
    wid = s * NC + c
    one16 = jnp.ones((16,), jnp.float32)
    zero16 = jnp.zeros((16,), jnp.float32)

    def _fill_ones(i, _):
        ones_v[pl.ds(i * 16, 16)] = one16
        return 0

    jax.lax.fori_loop(0, K // 16, _fill_ones, 0)

    def _fill_zero(i, _):
        zbuf[pl.ds(i * 16, 16)] = zero16
        return 0

    jax.lax.fori_loop(0, APT // 16, _fill_zero, 0)
    pltpu.sync_copy(zbuf, acc.at[pl.ds(s * APT, APT)])
    plsc.subcore_barrier()
    pltpu.sync_copy(dst_hbm.at[wid], dst_v)

    def _step(j, _):
        pltpu.sync_copy(ones_v, acc.at[dst_v.at[j]], add=True)
        return 0

    jax.lax.fori_loop(0, R0, _step, 0)
    plsc.subcore_barrier()
    pltpu.sync_copy(acc.at[pl.ds(s * APT, APT)],
                    out_hbm.at[c, pl.ds(s * APT, APT)])


HH = H // 2


NBUF = 4
R0 = 128
R1 = 32
NSLAB = NW


def _sc_scatter_body(ypl_hbm, ypr_hbm, src_hbm, dst_hbm, out_hbm,
                     src_v, dst_v, rows, gsems, ssems, acc):
    c = jax.lax.axis_index("c")
    s = jax.lax.axis_index("s")
    wid = c * NS + s
    zero16 = jnp.zeros((16,), jnp.float32)
    rc = jnp.where(c == 0, R0, R1)
    nsteps = jnp.where(c == 0, R0 // NBUF, R1 // NBUF)

    def _zrow(r, _):
        for m in range(HH // 16):
            rows[0][r, pl.ds(m * 16, 16)] = zero16
        return 0

    pltpu.sync_copy(src_hbm.at[wid], src_v)
    pltpu.sync_copy(dst_hbm.at[wid], dst_v)
    for half, tab in ((0, ypl_hbm), (1, ypr_hbm)):
        jax.lax.fori_loop(0, K, _zrow, 0)
        for q in range(APT // K):
            pltpu.sync_copy(rows[0], acc.at[pl.ds(s * APT + q * K, K)])
        plsc.subcore_barrier()

        def _gather(j, b, tab=tab):
            return pltpu.async_copy(tab.at[src_v.at[j]], rows[b], gsems[b])

        def _scatter(j, b):
            return pltpu.async_copy(rows[b], acc.at[dst_v.at[j]], ssems[b],
                                    add=True)

        for b in range(NBUF):
            _gather(b, b)

        def _step(g, _, tab=tab):
            base = g * NBUF
            for pair in range(NBUF // 2):
                bs = (2 * pair, 2 * pair + 1)
                for b in bs:
                    j = base + b
                    pltpu.make_async_copy(tab.at[src_v.at[j]], rows[b],
                                          gsems[b]).wait()
                    _scatter(j, b)
                for b in bs:
                    j = base + b
                    pltpu.make_async_copy(rows[b], acc.at[dst_v.at[j]],
                                          ssems[b]).wait()

                    @pl.when(j + NBUF < rc)
                    def _(j=j, b=b):
                        _gather(j + NBUF, b)
            return 0

        jax.lax.fori_loop(0, nsteps, _step, 0)
        plsc.subcore_barrier()
        pltpu.sync_copy(acc.at[pl.ds(s * APT, APT)],
                        out_hbm.at[c, half, pl.ds(s * APT, APT)])


def _sc_pool_body(h_hbm, b_hbm, mxo_hbm, smo_hbm, cno_hbm,
                  hbuf, bseg, mx, sm, cn):
    c = jax.lax.axis_index("c")
    s = jax.lax.axis_index("s")
    wid = s * NC + c
    neg16 = jnp.full((16,), -jnp.inf, jnp.float32)
    zero16 = jnp.zeros((16,), jnp.float32)
    one16 = jnp.ones((16,), jnp.float32)

    def _init(r, _):
        for m in range(H // 16):
            mx[r, pl.ds(m * 16, 16)] = neg16
            sm[r, pl.ds(m * 16, 16)] = zero16
        cn[r, pl.ds(0, 16)] = zero16
        return 0

    jax.lax.fori_loop(0, G + 8, _init, 0)
    pltpu.sync_copy(b_hbm.at[pl.ds(wid * RPT, RPT)], bseg)

    def _chunk(q, _):
        pltpu.sync_copy(h_hbm.at[pl.ds(wid * RPT + q * 64, 64)], hbuf)

        def _grp(gi, _):
            sv = bseg[pl.ds(q * 64 + gi * 16, 16)]
            for t in range(16):
                seg = sv[t]
                r = gi * 16 + t
                for m in range(H // 16):
                    sl = pl.ds(m * 16, 16)
                    hv = hbuf[r, sl]
                    mx[seg, sl] = jnp.maximum(mx[seg, sl], hv)
                    sm[seg, sl] = sm[seg, sl] + hv
                cn[seg, pl.ds(0, 16)] = cn[seg, pl.ds(0, 16)] + one16
            return 0

        jax.lax.fori_loop(0, 4, _grp, 0)
        return 0

    jax.lax.fori_loop(0, RPT // 64, _chunk, 0)
    pltpu.sync_copy(mx.at[pl.ds(0, G)], mxo_hbm.at[wid])
    pltpu.sync_copy(sm.at[pl.ds(0, G)], smo_hbm.at[wid])
    pltpu.sync_copy(cn.at[pl.ds(0, G)], cno_hbm.at[wid])


@functools.cache
def _sc_kernels():
    mesh = plsc.VectorSubcoreMesh(core_axis_name="c", subcore_axis_name="s",
                                  num_cores=NC, num_subcores=NS)
    deg = pl.kernel(
        _sc_deg_body,
        out_type=jax.ShapeDtypeStruct((NC, NPAD), jnp.float32),
        mesh=mesh,
        scratch_types=[
            pltpu.VMEM((R0, K), jnp.int32),
            pltpu.VMEM((K,), jnp.float32),
            pltpu.VMEM((APT,), jnp.float32),
            pltpu.VMEM_SHARED((NPAD,), jnp.float32),
        ],
    )
    scatter = pl.kernel(
        _sc_scatter_body,
        out_type=jax.ShapeDtypeStruct((NC, 2, NPAD, HH), jnp.float32),
        mesh=mesh,
        compiler_params=pltpu.CompilerParams(use_tc_tiling_on_sc=False),
        scratch_types=[
            pltpu.VMEM((R0, K), jnp.int32),
            pltpu.VMEM((R0, K), jnp.int32),
            [pltpu.VMEM((K, HH), jnp.float32) for _ in range(NBUF)],
            [pltpu.SemaphoreType.DMA for _ in range(NBUF)],
            [pltpu.SemaphoreType.DMA for _ in range(NBUF)],
            pltpu.VMEM_SHARED((NPAD, HH), jnp.float32),
        ],
    )
    pool = pl.kernel(
        _sc_pool_body,
        out_type=[
            jax.ShapeDtypeStruct((NW, G, H), jnp.float32),
            jax.ShapeDtypeStruct((NW, G, H), jnp.float32),
            jax.ShapeDtypeStruct((NW, G, 16), jnp.float32),
        ],
        mesh=mesh,
        scratch_types=[
            pltpu.VMEM((64, H), jnp.float32),
            pltpu.VMEM((RPT,), jnp.int32),
            pltpu.VMEM((G + 8, H), jnp.float32),
            pltpu.VMEM((G + 8, H), jnp.float32),
            pltpu.VMEM((G + 8, 16), jnp.float32),
        ],
    )
    return deg, scatter, pool


def _sc_deg(dst3):
    return _sc_kernels()[0](dst3)


def _sc_scatter(ypl, ypr, src3, dst3):
    return _sc_kernels()[1](ypl, ypr, src3, dst3)


def _sc_pool(h, batchp):
    return _sc_kernels()[2](h, batchp)


BLK = 1024
GRID = NPAD // BLK


def _layer0_body(x_ref, d0_ref, d1_ref, w_ref, ypl_ref, ypr_ref, dinv_ref):
    dinv = jax.lax.rsqrt(1.0 + d0_ref[...] + d1_ref[...])
    dinv_ref[...] = dinv
    y = jnp.dot(x_ref[...], w_ref[...], preferred_element_type=jnp.float32)
    yp = y * dinv
    ypl_ref[...] = yp[:, :HH]
    ypr_ref[...] = yp[:, HH:]


_tc_layer0 = pl.pallas_call(
    _layer0_body,
    grid=(GRID,),
    in_specs=[
        pl.BlockSpec((BLK, D), lambda i: (i, 0)),
        pl.BlockSpec((BLK, 1), lambda i: (i, 0)),
        pl.BlockSpec((BLK, 1), lambda i: (i, 0)),
        pl.BlockSpec((D, H), lambda i: (0, 0)),
    ],
    out_specs=[
        pl.BlockSpec((BLK, HH), lambda i: (i, 0)),
        pl.BlockSpec((BLK, HH), lambda i: (i, 0)),
        pl.BlockSpec((BLK, 1), lambda i: (i, 0)),
    ],
    out_shape=[
        jax.ShapeDtypeStruct((NPAD, HH), jnp.float32),
        jax.ShapeDtypeStruct((NPAD, HH), jnp.float32),
        jax.ShapeDtypeStruct((NPAD, 1), jnp.float32),
    ],
)


def _state(p_ref, yppl_ref, yppr_ref, dinv_ref, b_ref, g_ref, bb_ref):
    accl = p_ref[0] + p_ref[2] + yppl_ref[...]
    accr = p_ref[1] + p_ref[3] + yppr_ref[...]
    agg = (dinv_ref[...] * jnp.concatenate([accl, accr], axis=1)
           + b_ref[...])
    return jnp.maximum(agg * BN_SCALE * g_ref[...] + bb_ref[...], 0.0)


def _layer_body(p_ref, yppl_ref, yppr_ref, dinv_ref, b_ref, g_ref, bb_ref,
                w_ref, ypl_ref, ypr_ref):
    st = _state(p_ref, yppl_ref, yppr_ref, dinv_ref, b_ref, g_ref, bb_ref)
    y = jnp.dot(st, w_ref[...], preferred_element_type=jnp.float32)
    yp = y * dinv_ref[...]
    ypl_ref[...] = yp[:, :HH]
    ypr_ref[...] = yp[:, HH:]


_tc_layer = pl.pallas_call(
    _layer_body,
    grid=(GRID,),
    in_specs=[
        pl.BlockSpec((4, BLK, HH), lambda i: (0, i, 0)),
        pl.BlockSpec((BLK, HH), lambda i: (i, 0)),
        pl.BlockSpec((BLK, HH), lambda i: (i, 0)),
        pl.BlockSpec((BLK, 1), lambda i: (i, 0)),
        pl.BlockSpec((1, H), lambda i: (0, 0)),
        pl.BlockSpec((1, H), lambda i: (0, 0)),
        pl.BlockSpec((1, H), lambda i: (0, 0)),
        pl.BlockSpec((H, H), lambda i: (0, 0)),
    ],
    out_specs=[
        pl.BlockSpec((BLK, HH), lambda i: (i, 0)),
        pl.BlockSpec((BLK, HH), lambda i: (i, 0)),
    ],
    out_shape=[
        jax.ShapeDtypeStruct((NPAD, HH), jnp.float32),
        jax.ShapeDtypeStruct((NPAD, HH), jnp.float32),
    ],
)


def _final_body(p_ref, yppl_ref, yppr_ref, dinv_ref, b_ref, g_ref, bb_ref,
                h_ref):
    h_ref[...] = _state(p_ref, yppl_ref, yppr_ref, dinv_ref, b_ref, g_ref,
                        bb_ref)


_tc_final = pl.pallas_call(
    _final_body,
    grid=(GRID,),
    in_specs=[
        pl.BlockSpec((4, BLK, HH), lambda i: (0, i, 0)),
        pl.BlockSpec((BLK, HH), lambda i: (i, 0)),
        pl.BlockSpec((BLK, HH), lambda i: (i, 0)),
        pl.BlockSpec((BLK, 1), lambda i: (i, 0)),
        pl.BlockSpec((1, H), lambda i: (0, 0)),
        pl.BlockSpec((1, H), lambda i: (0, 0)),
        pl.BlockSpec((1, H), lambda i: (0, 0)),
    ],
    out_specs=pl.BlockSpec((BLK, H), lambda i: (i, 0)),
    out_shape=jax.ShapeDtypeStruct((NPAD, H), jnp.float32),
)


def _head_body(sump_ref, cntp_ref, maxp_ref, gfin_ref,
               gw1, gb1, gg1, gbb1, gw2, gb2, gg2, gbb2,
               fw1, fb1, fg1, fbb1, fw2, fb2, fg2, fbb2, fw3, fb3,
               z_ref):
    cnt = jnp.sum(cntp_ref[...], axis=0)[:, 0:1]
    ssum = jnp.sum(sump_ref[...], axis=0)
    smax = jnp.max(maxp_ref[...], axis=0)
    x1 = ssum / jnp.maximum(cnt, 1.0)
    x2 = jnp.where(cnt > 0.0, smax, 0.0)

    def bnrelu(t, g, bb):
        return jnp.maximum(t * BN_SCALE * g[...] + bb[...], 0.0)

    gf = jnp.dot(gfin_ref[...], gw1[...],
                 preferred_element_type=jnp.float32) + gb1[...]
    gf = bnrelu(gf, gg1, gbb1)
    gf = jnp.dot(gf, gw2[...], preferred_element_type=jnp.float32) + gb2[...]
    gf = bnrelu(gf, gg2, gbb2)
    fused = jnp.concatenate([x1, x2, ssum, gf], axis=1)
    z = jnp.dot(fused, fw1[...], preferred_element_type=jnp.float32) + fb1[...]
    z = bnrelu(z, fg1, fbb1)
    z = jnp.dot(z, fw2[...], preferred_element_type=jnp.float32) + fb2[...]
    z = bnrelu(z, fg2, fbb2)
    z_ref[...] = jnp.dot(z, fw3[...],
                         preferred_element_type=jnp.float32) + fb3[...]


_tc_head = pl.pallas_call(
    _head_body,
    out_shape=jax.ShapeDtypeStruct((G, 1), jnp.float32),
)


def kernel(x, edge_index, batch, graph_features, params):
    src = edge_index[0].astype(jnp.int32)
    dst = edge_index[1].astype(jnp.int32)
    e0 = NS * R0 * K
    e1r = NS * R1 * K
    ndead = NPAD - N
    def dead(n):
        return N + jnp.arange(n, dtype=jnp.int32) % ndead
    def slabs(v, pad1, pad2):
        p0 = v[:e0].reshape(NS, R0, K)
        p1 = jnp.concatenate([v[e0:], pad1]).reshape(NS, R1, K)
        p1 = jnp.concatenate([p1, pad2.reshape(NS, R0 - R1, K)], axis=1)
        return jnp.concatenate([p0.reshape(NS, R0, K), p1], axis=0)
    npad1 = e1r - (E - e0)
    npad2 = NS * (R0 - R1) * K
    src3 = slabs(src, jnp.zeros((npad1,), jnp.int32),
                 jnp.zeros((npad2,), jnp.int32))
    dst3 = slabs(dst, dead(npad1), dead(npad2))
    batchp = jnp.concatenate(
        [batch.astype(jnp.int32), jnp.full((NPAD - N,), G, jnp.int32)])
    xp = jnp.pad(x, ((0, NPAD - N), (0, 0)))

    def row(v):
        return v.reshape(1, -1)

    degp = _sc_deg(dst3)
    d0 = degp[0].reshape(NPAD, 1)
    d1 = degp[1].reshape(NPAD, 1)
    ypl, ypr, dinv = _tc_layer0(xp, d0, d1, params["gcn_w0"])
    for l in range(1, NLAYERS):
        p = _sc_scatter(ypl, ypr, src3, dst3).reshape(4, NPAD, HH)
        ypl, ypr = _tc_layer(p, ypl, ypr, dinv,
                             row(params[f"gcn_b{l-1}"]),
                             row(params[f"bn_g{l-1}"]),
                             row(params[f"bn_b{l-1}"]), params[f"gcn_w{l}"])
    p = _sc_scatter(ypl, ypr, src3, dst3).reshape(4, NPAD, HH)
    h = _tc_final(p, ypl, ypr, dinv,
                  row(params["gcn_b3"]), row(params["bn_g3"]),
                  row(params["bn_b3"]))
    maxp, sump, cntp = _sc_pool(h, batchp)
    z = _tc_head(sump, cntp, maxp, graph_features,
                 params["gm_w1"], row(params["gm_b1"]),
                 row(params["gm_g1"]), row(params["gm_bb1"]),
                 params["gm_w2"], row(params["gm_b2"]),
                 row(params["gm_g2"]), row(params["gm_bb2"]),
                 params["f_w1"], row(params["f_b1"]),
                 row(params["f_g1"]), row(params["f_bb1"]),
                 params["f_w2"], row(params["f_b2"]),
                 row(params["f_g2"]), row(params["f_bb2"]),
                 params["f_w3"], row(params["f_b3"]))
    return z


# speedup vs baseline: 8.0984x; 1.2028x over previous
import functools

import jax
import jax.numpy as jnp
from jax.experimental import pallas as pl
from jax.experimental.pallas import tpu as pltpu
from jax.experimental.pallas import tpu_sc as plsc

N = 10000
E = 320000
D = 128
H = 128
G = 64
GF = 32
NLAYERS = 4
BN_SCALE = (1.0 + 1e-5) ** -0.5

NC = 2
NS = 16
NW = NC * NS
K = 128
CH = 80
E_PAD = NW * CH * K
NPAD = 10240
APT = NPAD // NS
RPT = NPAD // NW


def _sc_deg_body(dst_hbm, out_hbm, dst_v, ones_v, zbuf, acc):
    c = jax.lax.axis_index("c")
    s = jax.lax.axis_index("s")
    wid = s * NC + c
    one16 = jnp.ones((16,), jnp.float32)
    zero16 = jnp.zeros((16,), jnp.float32)

    def _fill_ones(i, _):
        ones_v[pl.ds(i * 16, 16)] = one16
        return 0

    jax.lax.fori_loop(0, K // 16, _fill_ones, 0)

    def _fill_zero(i, _):
        zbuf[pl.ds(i * 16, 16)] = zero16
        return 0

    jax.lax.fori_loop(0, APT // 16, _fill_zero, 0)
    pltpu.sync_copy(zbuf, acc.at[pl.ds(s * APT, APT)])
    plsc.subcore_barrier()
    pltpu.sync_copy(dst_hbm.at[wid], dst_v)

    def _step(j, _):
        pltpu.sync_copy(ones_v, acc.at[dst_v.at[j]], add=True)
        return 0

    jax.lax.fori_loop(0, R0, _step, 0)
    plsc.subcore_barrier()
    pltpu.sync_copy(acc.at[pl.ds(s * APT, APT)],
                    out_hbm.at[c, pl.ds(s * APT, APT)])


HH = H // 2


NBUF = 4
R0 = 152
R1 = 8
NSLAB = NW


def _sc_scatter_body(ypl_hbm, ypr_hbm, src_hbm, dst_hbm, out_hbm,
                     src_v, dst_v, rows, gsems, ssems, acc):
    c = jax.lax.axis_index("c")
    s = jax.lax.axis_index("s")
    wid = c * NS + s
    zero16 = jnp.zeros((16,), jnp.float32)
    rc = jnp.where(c == 0, R0, R1)
    nsteps = jnp.where(c == 0, R0 // NBUF, R1 // NBUF)

    def _zrow(r, _):
        for m in range(HH // 16):
            rows[0][r, pl.ds(m * 16, 16)] = zero16
        return 0

    pltpu.sync_copy(src_hbm.at[wid], src_v)
    pltpu.sync_copy(dst_hbm.at[wid], dst_v)
    for half, tab in ((0, ypl_hbm), (1, ypr_hbm)):
        jax.lax.fori_loop(0, K, _zrow, 0)
        for q in range(APT // K):
            pltpu.sync_copy(rows[0], acc.at[pl.ds(s * APT + q * K, K)])
        plsc.subcore_barrier()

        def _gather(j, b, tab=tab):
            return pltpu.async_copy(tab.at[src_v.at[j]], rows[b], gsems[b])

        def _scatter(j, b):
            return pltpu.async_copy(rows[b], acc.at[dst_v.at[j]], ssems[b],
                                    add=True)

        for b in range(NBUF):
            _gather(b, b)

        def _step(g, _, tab=tab):
            base = g * NBUF
            for pair in range(NBUF // 2):
                bs = (2 * pair, 2 * pair + 1)
                for b in bs:
                    j = base + b
                    pltpu.make_async_copy(tab.at[src_v.at[j]], rows[b],
                                          gsems[b]).wait()
                    _scatter(j, b)
                for b in bs:
                    j = base + b
                    pltpu.make_async_copy(rows[b], acc.at[dst_v.at[j]],
                                          ssems[b]).wait()

                    @pl.when(j + NBUF < rc)
                    def _(j=j, b=b):
                        _gather(j + NBUF, b)
            return 0

        jax.lax.fori_loop(0, nsteps, _step, 0)
        plsc.subcore_barrier()
        pltpu.sync_copy(acc.at[pl.ds(s * APT, APT)],
                        out_hbm.at[c, half, pl.ds(s * APT, APT)])


def _sc_pool_body(h_hbm, b_hbm, mxo_hbm, smo_hbm, cno_hbm,
                  hbuf, bseg, mx, sm, cn):
    c = jax.lax.axis_index("c")
    s = jax.lax.axis_index("s")
    wid = s * NC + c
    neg16 = jnp.full((16,), -jnp.inf, jnp.float32)
    zero16 = jnp.zeros((16,), jnp.float32)
    one16 = jnp.ones((16,), jnp.float32)

    def _init(r, _):
        for m in range(H // 16):
            mx[r, pl.ds(m * 16, 16)] = neg16
            sm[r, pl.ds(m * 16, 16)] = zero16
        cn[r, pl.ds(0, 16)] = zero16
        return 0

    jax.lax.fori_loop(0, G + 8, _init, 0)
    pltpu.sync_copy(b_hbm.at[pl.ds(wid * RPT, RPT)], bseg)

    def _chunk(q, _):
        pltpu.sync_copy(h_hbm.at[pl.ds(wid * RPT + q * 64, 64)], hbuf)

        def _grp(gi, _):
            sv = bseg[pl.ds(q * 64 + gi * 16, 16)]
            for t in range(16):
                seg = sv[t]
                r = gi * 16 + t
                for m in range(H // 16):
                    sl = pl.ds(m * 16, 16)
                    hv = hbuf[r, sl]
                    mx[seg, sl] = jnp.maximum(mx[seg, sl], hv)
                    sm[seg, sl] = sm[seg, sl] + hv
                cn[seg, pl.ds(0, 16)] = cn[seg, pl.ds(0, 16)] + one16
            return 0

        jax.lax.fori_loop(0, 4, _grp, 0)
        return 0

    jax.lax.fori_loop(0, RPT // 64, _chunk, 0)
    pltpu.sync_copy(mx.at[pl.ds(0, G)], mxo_hbm.at[wid])
    pltpu.sync_copy(sm.at[pl.ds(0, G)], smo_hbm.at[wid])
    pltpu.sync_copy(cn.at[pl.ds(0, G)], cno_hbm.at[wid])


@functools.cache
def _sc_kernels():
    mesh = plsc.VectorSubcoreMesh(core_axis_name="c", subcore_axis_name="s",
                                  num_cores=NC, num_subcores=NS)
    deg = pl.kernel(
        _sc_deg_body,
        out_type=jax.ShapeDtypeStruct((NC, NPAD), jnp.float32),
        mesh=mesh,
        scratch_types=[
            pltpu.VMEM((R0, K), jnp.int32),
            pltpu.VMEM((K,), jnp.float32),
            pltpu.VMEM((APT,), jnp.float32),
            pltpu.VMEM_SHARED((NPAD,), jnp.float32),
        ],
    )
    scatter = pl.kernel(
        _sc_scatter_body,
        out_type=jax.ShapeDtypeStruct((NC, 2, NPAD, HH), jnp.float32),
        mesh=mesh,
        compiler_params=pltpu.CompilerParams(use_tc_tiling_on_sc=False),
        scratch_types=[
            pltpu.VMEM((R0, K), jnp.int32),
            pltpu.VMEM((R0, K), jnp.int32),
            [pltpu.VMEM((K, HH), jnp.float32) for _ in range(NBUF)],
            [pltpu.SemaphoreType.DMA for _ in range(NBUF)],
            [pltpu.SemaphoreType.DMA for _ in range(NBUF)],
            pltpu.VMEM_SHARED((NPAD, HH), jnp.float32),
        ],
    )
    pool = pl.kernel(
        _sc_pool_body,
        out_type=[
            jax.ShapeDtypeStruct((NW, G, H), jnp.float32),
            jax.ShapeDtypeStruct((NW, G, H), jnp.float32),
            jax.ShapeDtypeStruct((NW, G, 16), jnp.float32),
        ],
        mesh=mesh,
        scratch_types=[
            pltpu.VMEM((64, H), jnp.float32),
            pltpu.VMEM((RPT,), jnp.int32),
            pltpu.VMEM((G + 8, H), jnp.float32),
            pltpu.VMEM((G + 8, H), jnp.float32),
            pltpu.VMEM((G + 8, 16), jnp.float32),
        ],
    )
    return deg, scatter, pool


def _sc_deg(dst3):
    return _sc_kernels()[0](dst3)


def _sc_scatter(ypl, ypr, src3, dst3):
    return _sc_kernels()[1](ypl, ypr, src3, dst3)


def _sc_pool(h, batchp):
    return _sc_kernels()[2](h, batchp)


BLK = 1024
GRID = NPAD // BLK


def _layer0_body(x_ref, d0_ref, d1_ref, w_ref, ypl_ref, ypr_ref, dinv_ref):
    dinv = jax.lax.rsqrt(1.0 + d0_ref[...] + d1_ref[...])
    dinv_ref[...] = dinv
    y = jnp.dot(x_ref[...], w_ref[...], preferred_element_type=jnp.float32)
    yp = y * dinv
    ypl_ref[...] = yp[:, :HH]
    ypr_ref[...] = yp[:, HH:]


_tc_layer0 = pl.pallas_call(
    _layer0_body,
    grid=(GRID,),
    in_specs=[
        pl.BlockSpec((BLK, D), lambda i: (i, 0)),
        pl.BlockSpec((BLK, 1), lambda i: (i, 0)),
        pl.BlockSpec((BLK, 1), lambda i: (i, 0)),
        pl.BlockSpec((D, H), lambda i: (0, 0)),
    ],
    out_specs=[
        pl.BlockSpec((BLK, HH), lambda i: (i, 0)),
        pl.BlockSpec((BLK, HH), lambda i: (i, 0)),
        pl.BlockSpec((BLK, 1), lambda i: (i, 0)),
    ],
    out_shape=[
        jax.ShapeDtypeStruct((NPAD, HH), jnp.float32),
        jax.ShapeDtypeStruct((NPAD, HH), jnp.float32),
        jax.ShapeDtypeStruct((NPAD, 1), jnp.float32),
    ],
)


def _state(p_ref, yppl_ref, yppr_ref, dinv_ref, b_ref, g_ref, bb_ref):
    accl = p_ref[0] + p_ref[2] + yppl_ref[...]
    accr = p_ref[1] + p_ref[3] + yppr_ref[...]
    agg = (dinv_ref[...] * jnp.concatenate([accl, accr], axis=1)
           + b_ref[...])
    return jnp.maximum(agg * BN_SCALE * g_ref[...] + bb_ref[...], 0.0)


def _layer_body(p_ref, yppl_ref, yppr_ref, dinv_ref, b_ref, g_ref, bb_ref,
                w_ref, ypl_ref, ypr_ref):
    st = _state(p_ref, yppl_ref, yppr_ref, dinv_ref, b_ref, g_ref, bb_ref)
    y = jnp.dot(st, w_ref[...], preferred_element_type=jnp.float32)
    yp = y * dinv_ref[...]
    ypl_ref[...] = yp[:, :HH]
    ypr_ref[...] = yp[:, HH:]


_tc_layer = pl.pallas_call(
    _layer_body,
    grid=(GRID,),
    in_specs=[
        pl.BlockSpec((4, BLK, HH), lambda i: (0, i, 0)),
        pl.BlockSpec((BLK, HH), lambda i: (i, 0)),
        pl.BlockSpec((BLK, HH), lambda i: (i, 0)),
        pl.BlockSpec((BLK, 1), lambda i: (i, 0)),
        pl.BlockSpec((1, H), lambda i: (0, 0)),
        pl.BlockSpec((1, H), lambda i: (0, 0)),
        pl.BlockSpec((1, H), lambda i: (0, 0)),
        pl.BlockSpec((H, H), lambda i: (0, 0)),
    ],
    out_specs=[
        pl.BlockSpec((BLK, HH), lambda i: (i, 0)),
        pl.BlockSpec((BLK, HH), lambda i: (i, 0)),
    ],
    out_shape=[
        jax.ShapeDtypeStruct((NPAD, HH), jnp.float32),
        jax.ShapeDtypeStruct((NPAD, HH), jnp.float32),
    ],
)


def _final_body(p_ref, yppl_ref, yppr_ref, dinv_ref, b_ref, g_ref, bb_ref,
                h_ref):
    h_ref[...] = _state(p_ref, yppl_ref, yppr_ref, dinv_ref, b_ref, g_ref,
                        bb_ref)


_tc_final = pl.pallas_call(
    _final_body,
    grid=(GRID,),
    in_specs=[
        pl.BlockSpec((4, BLK, HH), lambda i: (0, i, 0)),
        pl.BlockSpec((BLK, HH), lambda i: (i, 0)),
        pl.BlockSpec((BLK, HH), lambda i: (i, 0)),
        pl.BlockSpec((BLK, 1), lambda i: (i, 0)),
        pl.BlockSpec((1, H), lambda i: (0, 0)),
        pl.BlockSpec((1, H), lambda i: (0, 0)),
        pl.BlockSpec((1, H), lambda i: (0, 0)),
    ],
    out_specs=pl.BlockSpec((BLK, H), lambda i: (i, 0)),
    out_shape=jax.ShapeDtypeStruct((NPAD, H), jnp.float32),
)


def _head_body(sump_ref, cntp_ref, maxp_ref, gfin_ref,
               gw1, gb1, gg1, gbb1, gw2, gb2, gg2, gbb2,
               fw1, fb1, fg1, fbb1, fw2, fb2, fg2, fbb2, fw3, fb3,
               z_ref):
    cnt = jnp.sum(cntp_ref[...], axis=0)[:, 0:1]
    ssum = jnp.sum(sump_ref[...], axis=0)
    smax = jnp.max(maxp_ref[...], axis=0)
    x1 = ssum / jnp.maximum(cnt, 1.0)
    x2 = jnp.where(cnt > 0.0, smax, 0.0)

    def bnrelu(t, g, bb):
        return jnp.maximum(t * BN_SCALE * g[...] + bb[...], 0.0)

    gf = jnp.dot(gfin_ref[...], gw1[...],
                 preferred_element_type=jnp.float32) + gb1[...]
    gf = bnrelu(gf, gg1, gbb1)
    gf = jnp.dot(gf, gw2[...], preferred_element_type=jnp.float32) + gb2[...]
    gf = bnrelu(gf, gg2, gbb2)
    fused = jnp.concatenate([x1, x2, ssum, gf], axis=1)
    z = jnp.dot(fused, fw1[...], preferred_element_type=jnp.float32) + fb1[...]
    z = bnrelu(z, fg1, fbb1)
    z = jnp.dot(z, fw2[...], preferred_element_type=jnp.float32) + fb2[...]
    z = bnrelu(z, fg2, fbb2)
    z_ref[...] = jnp.dot(z, fw3[...],
                         preferred_element_type=jnp.float32) + fb3[...]


_tc_head = pl.pallas_call(
    _head_body,
    out_shape=jax.ShapeDtypeStruct((G, 1), jnp.float32),
)


def kernel(x, edge_index, batch, graph_features, params):
    src = edge_index[0].astype(jnp.int32)
    dst = edge_index[1].astype(jnp.int32)
    e0 = NS * R0 * K
    e1r = NS * R1 * K
    ndead = NPAD - N
    def dead(n):
        return N + jnp.arange(n, dtype=jnp.int32) % ndead
    def slabs(v, pad1, pad2):
        p0 = v[:e0].reshape(NS, R0, K)
        p1 = jnp.concatenate([v[e0:], pad1]).reshape(NS, R1, K)
        p1 = jnp.concatenate([p1, pad2.reshape(NS, R0 - R1, K)], axis=1)
        return jnp.concatenate([p0.reshape(NS, R0, K), p1], axis=0)
    npad1 = e1r - (E - e0)
    npad2 = NS * (R0 - R1) * K
    src3 = slabs(src, jnp.zeros((npad1,), jnp.int32),
                 jnp.zeros((npad2,), jnp.int32))
    dst3 = slabs(dst, dead(npad1), dead(npad2))
    batchp = jnp.concatenate(
        [batch.astype(jnp.int32), jnp.full((NPAD - N,), G, jnp.int32)])
    xp = jnp.pad(x, ((0, NPAD - N), (0, 0)))

    def row(v):
        return v.reshape(1, -1)

    degp = _sc_deg(dst3)
    d0 = degp[0].reshape(NPAD, 1)
    d1 = degp[1].reshape(NPAD, 1)
    ypl, ypr, dinv = _tc_layer0(xp, d0, d1, params["gcn_w0"])
    for l in range(1, NLAYERS):
        p = _sc_scatter(ypl, ypr, src3, dst3).reshape(4, NPAD, HH)
        ypl, ypr = _tc_layer(p, ypl, ypr, dinv,
                             row(params[f"gcn_b{l-1}"]),
                             row(params[f"bn_g{l-1}"]),
                             row(params[f"bn_b{l-1}"]), params[f"gcn_w{l}"])
    p = _sc_scatter(ypl, ypr, src3, dst3).reshape(4, NPAD, HH)
    h = _tc_final(p, ypl, ypr, dinv,
                  row(params["gcn_b3"]), row(params["bn_g3"]),
                  row(params["bn_b3"]))
    maxp, sump, cntp = _sc_pool(h, batchp)
    z = _tc_head(sump, cntp, maxp, graph_features,
                 params["gm_w1"], row(params["gm_b1"]),
                 row(params["gm_g1"]), row(params["gm_bb1"]),
                 params["gm_w2"], row(params["gm_b2"]),
                 row(params["gm_g2"]), row(params["gm_bb2"]),
                 params["f_w1"], row(params["f_b1"]),
                 row(params["f_g1"]), row(params["f_bb1"]),
                 params["f_w2"], row(params["f_b2"]),
                 row(params["f_g2"]), row(params["f_bb2"]),
                 params["f_w3"], row(params["f_b3"]))
    return z
